# baseline (speedup 1.0000x reference)
import functools

import jax
import jax.numpy as jnp
from jax import lax
from jax.experimental import pallas as pl
from jax.experimental.pallas import tpu as pltpu
from jax.experimental.pallas import tpu_sc as plsc

N = 10000
E = 320000
H = 128
NORM_FACTOR = 100.0
NORM_CONST = 1.0

NC = 2
NS = 16
NW = NC * NS
EPW = E // NW
NSLICE = 2
ES = E // NSLICE
ESW = ES // NW
CHUNK = 400
NPT = 624
NREM = N - NS * NPT

f32 = jnp.float32
bf16 = jnp.bfloat16
i32 = jnp.int32


def _mesh():
    return plsc.VectorSubcoreMesh(core_axis_name="c", subcore_axis_name="s")


def _gather_add(A, B, row, col, sl):
    C = 200
    NCHUNK = ESW // C
    NPAIR = (NCHUNK - 1) // 2

    @functools.partial(
        pl.kernel,
        mesh=_mesh(),
        compiler_params=pltpu.CompilerParams(needs_layout_passes=False),
        out_type=jax.ShapeDtypeStruct((ES, H), f32),
        scratch_types=[
            pltpu.VMEM((ESW,), i32), pltpu.VMEM((ESW,), i32),
            pltpu.VMEM((C, H), f32), pltpu.VMEM((C, H), f32),
            pltpu.VMEM((C, H), f32), pltpu.VMEM((C, H), f32),
            pltpu.SemaphoreType.DMA, pltpu.SemaphoreType.DMA,
        ],
    )
    def k(a_hbm, b_hbm, row_hbm, col_hbm, g_hbm,
          rv, cv, a0, b0, a1, b1, gsem, wsem):
        wid = lax.axis_index("s") * NC + lax.axis_index("c")
        base = wid * ESW
        gbase = sl * ES + base
        pltpu.sync_copy(row_hbm.at[pl.ds(gbase, ESW)], rv)
        pltpu.sync_copy(col_hbm.at[pl.ds(gbase, ESW)], cv)

        def addloop(abuf, bbuf):
            def arow(r, c2):
                for j in range(H // 16):
                    slc = pl.ds(j * 16, 16)
                    abuf[r, slc] = abuf[r, slc] + bbuf[r, slc]
                return c2

            lax.fori_loop(0, C, arow, 0)

        def body(i, carry):
            loc0 = (2 * i) * C
            loc1 = loc0 + C
            off0 = base + loc0
            off1 = base + loc1

            @pl.when(i > 0)
            def _():
                pltpu.make_async_copy(a0, g_hbm.at[pl.ds(off0, C)], wsem).wait()
                pltpu.make_async_copy(a1, g_hbm.at[pl.ds(off1, C)], wsem).wait()

            g0a = pltpu.async_copy(a_hbm.at[rv.at[pl.ds(loc0, C)]], a0, gsem)
            g0b = pltpu.async_copy(b_hbm.at[cv.at[pl.ds(loc0, C)]], b0, gsem)
            g0a.wait()
            g0b.wait()
            g1a = pltpu.async_copy(a_hbm.at[rv.at[pl.ds(loc1, C)]], a1, gsem)
            g1b = pltpu.async_copy(b_hbm.at[cv.at[pl.ds(loc1, C)]], b1, gsem)
            addloop(a0, b0)
            pltpu.async_copy(a0, g_hbm.at[pl.ds(off0, C)], wsem)
            g1a.wait()
            g1b.wait()
            addloop(a1, b1)
            pltpu.async_copy(a1, g_hbm.at[pl.ds(off1, C)], wsem)
            return carry

        lax.fori_loop(0, NPAIR, body, 0)
        tloc = (2 * NPAIR) * C
        pltpu.make_async_copy(a0, g_hbm.at[pl.ds(base, C)], wsem).wait()
        ta = pltpu.async_copy(a_hbm.at[rv.at[pl.ds(tloc, C)]], a0, gsem)
        tb = pltpu.async_copy(b_hbm.at[cv.at[pl.ds(tloc, C)]], b0, gsem)
        ta.wait()
        tb.wait()
        addloop(a0, b0)
        pltpu.sync_copy(a0, g_hbm.at[pl.ds(base + tloc, C)])
        pltpu.make_async_copy(a1, g_hbm.at[pl.ds(base, C)], wsem).wait()

    return k(A, B, row, col)


def _radial_kernel(xs0, xs1, xs2, row, col):
    @functools.partial(
        pl.kernel,
        mesh=_mesh(),
        compiler_params=pltpu.CompilerParams(needs_layout_passes=False),
        out_type=jax.ShapeDtypeStruct((E,), f32),
        scratch_types=[
            pltpu.VMEM((N,), f32),
            pltpu.VMEM((N,), f32),
            pltpu.VMEM((N,), f32),
            pltpu.VMEM((CHUNK,), i32),
            pltpu.VMEM((CHUNK,), i32),
            pltpu.VMEM((CHUNK,), f32),
        ],
    )
    def k(x0_hbm, x1_hbm, x2_hbm, row_hbm, col_hbm, r_hbm,
          t0, t1, t2, rv, cv, rbuf):
        wid = lax.axis_index("s") * NC + lax.axis_index("c")
        pltpu.sync_copy(x0_hbm, t0)
        pltpu.sync_copy(x1_hbm, t1)
        pltpu.sync_copy(x2_hbm, t2)
        base = wid * EPW

        def body(i, carry):
            off = base + i * CHUNK
            pltpu.sync_copy(row_hbm.at[pl.ds(off, CHUNK)], rv)
            pltpu.sync_copy(col_hbm.at[pl.ds(off, CHUNK)], cv)

            def grp(j, c2):
                ridx = rv[pl.ds(j * 16, 16)]
                cidx = cv[pl.ds(j * 16, 16)]
                d0 = plsc.load_gather(t0, [ridx]) - plsc.load_gather(t0, [cidx])
                d1 = plsc.load_gather(t1, [ridx]) - plsc.load_gather(t1, [cidx])
                d2 = plsc.load_gather(t2, [ridx]) - plsc.load_gather(t2, [cidx])
                rbuf[pl.ds(j * 16, 16)] = d0 * d0 + d1 * d1 + d2 * d2
                return c2

            lax.fori_loop(0, CHUNK // 16, grp, 0)
            pltpu.sync_copy(rbuf, r_hbm.at[pl.ds(off, CHUNK)])
            return carry

        lax.fori_loop(0, EPW // CHUNK, body, 0)

    return k(xs0, xs1, xs2, row, col)


def _eq_scatter(s, row, col, xs0, xs1, xs2, zeros_n):
    @functools.partial(
        pl.kernel,
        mesh=_mesh(),
        compiler_params=pltpu.CompilerParams(needs_layout_passes=False),
        out_type=[jax.ShapeDtypeStruct((NC * N,), f32)] * 3,
        scratch_types=[
            pltpu.VMEM((N,), f32),
            pltpu.VMEM((N,), f32),
            pltpu.VMEM((N,), f32),
            pltpu.VMEM((CHUNK,), i32),
            pltpu.VMEM((CHUNK,), i32),
            pltpu.VMEM((CHUNK,), f32),
            pltpu.VMEM((CHUNK,), f32),
            pltpu.VMEM((CHUNK,), f32),
            pltpu.VMEM((CHUNK,), f32),
            pltpu.VMEM((N,), f32),
            pltpu.VMEM_SHARED((N,), f32),
            pltpu.VMEM_SHARED((N,), f32),
            pltpu.VMEM_SHARED((N,), f32),
        ],
    )
    def k(s_hbm, row_hbm, col_hbm, x0_hbm, x1_hbm, x2_hbm, z_hbm,
          o0_hbm, o1_hbm, o2_hbm,
          t0, t1, t2, rv, cv, sv, f0, f1, f2, stage, acc0, acc1, acc2):
        cid = lax.axis_index("c")
        sid = lax.axis_index("s")
        wid = sid * NC + cid
        pltpu.sync_copy(x0_hbm, t0)
        pltpu.sync_copy(x1_hbm, t1)
        pltpu.sync_copy(x2_hbm, t2)

        accs = [acc0, acc1, acc2]
        for kk in range(3):
            @pl.when(sid == kk)
            def _(kk=kk):
                pltpu.sync_copy(z_hbm, stage)
                pltpu.sync_copy(stage, accs[kk])

        plsc.subcore_barrier()
        base = wid * EPW

        def body(i, carry):
            off = base + i * CHUNK
            pltpu.sync_copy(row_hbm.at[pl.ds(off, CHUNK)], rv)
            pltpu.sync_copy(col_hbm.at[pl.ds(off, CHUNK)], cv)
            pltpu.sync_copy(s_hbm.at[pl.ds(off, CHUNK)], sv)

            def grp(j, c2):
                sl = pl.ds(j * 16, 16)
                ridx = rv[sl]
                cidx = cv[sl]
                se = sv[sl]
                f0[sl] = (plsc.load_gather(t0, [ridx])
                          - plsc.load_gather(t0, [cidx])) * se
                f1[sl] = (plsc.load_gather(t1, [ridx])
                          - plsc.load_gather(t1, [cidx])) * se
                f2[sl] = (plsc.load_gather(t2, [ridx])
                          - plsc.load_gather(t2, [cidx])) * se
                return c2

            lax.fori_loop(0, CHUNK // 16, grp, 0)
            pltpu.sync_copy(f0, acc0.at[rv], add=True)
            pltpu.sync_copy(f1, acc1.at[rv], add=True)
            pltpu.sync_copy(f2, acc2.at[rv], add=True)
            return carry

        lax.fori_loop(0, EPW // CHUNK, body, 0)
        plsc.subcore_barrier()

        outs = [o0_hbm, o1_hbm, o2_hbm]
        for kk in range(3):
            @pl.when(sid == kk)
            def _(kk=kk):
                pltpu.sync_copy(accs[kk], stage)
                pltpu.sync_copy(stage, outs[kk].at[pl.ds(cid * N, N)])

    return k(s, row, col, xs0, xs1, xs2, zeros_n)


def _scatter_add(vals0, vals1, row, zeros, D=H, CS=40):
    NPAIR = (ESW // CS - 1) // 2

    @functools.partial(
        pl.kernel,
        mesh=_mesh(),
        out_type=jax.ShapeDtypeStruct((NC, N, D), f32),
        scratch_types=[
            pltpu.VMEM((CS,), i32),
            pltpu.VMEM((CS,), i32),
            pltpu.VMEM((CS, D), f32),
            pltpu.VMEM((CS, D), f32),
            pltpu.VMEM_SHARED((N, D), f32),
            pltpu.SemaphoreType.DMA,
        ],
    )
    def k(vals0_hbm, vals1_hbm, row_hbm, zeros_hbm, out_hbm,
          i0, i1, v0, v1, acc, sem):
        cid = lax.axis_index("c")
        sid = lax.axis_index("s")
        wid = sid * NC + cid
        pltpu.sync_copy(zeros_hbm.at[pl.ds(sid * NPT, NPT)],
                        acc.at[pl.ds(sid * NPT, NPT)])

        @pl.when(sid == NS - 1)
        def _():
            pltpu.sync_copy(zeros_hbm.at[pl.ds(NS * NPT, NREM)],
                            acc.at[pl.ds(NS * NPT, NREM)])

        plsc.subcore_barrier()
        base = wid * ESW

        def run_slice(vals_hbm, gbase):
            pltpu.sync_copy(row_hbm.at[pl.ds(gbase, CS)], i0)
            pltpu.sync_copy(vals_hbm.at[pl.ds(base, CS)], v0)

            def body(i, carry):
                c1 = (2 * i + 1) * CS
                c2 = c1 + CS
                l1a = pltpu.async_copy(
                    row_hbm.at[pl.ds(gbase + c1, CS)], i1, sem)
                l1b = pltpu.async_copy(
                    vals_hbm.at[pl.ds(base + c1, CS)], v1, sem)
                pltpu.sync_copy(v0, acc.at[i0], add=True)
                l1a.wait()
                l1b.wait()
                l0a = pltpu.async_copy(
                    row_hbm.at[pl.ds(gbase + c2, CS)], i0, sem)
                l0b = pltpu.async_copy(
                    vals_hbm.at[pl.ds(base + c2, CS)], v0, sem)
                pltpu.sync_copy(v1, acc.at[i1], add=True)
                l0a.wait()
                l0b.wait()
                return carry

            lax.fori_loop(0, NPAIR, body, 0)
            pltpu.sync_copy(v0, acc.at[i0], add=True)

        run_slice(vals0_hbm, base)
        run_slice(vals1_hbm, ES + base)
        plsc.subcore_barrier()
        pltpu.sync_copy(acc.at[pl.ds(sid * NPT, NPT)],
                        out_hbm.at[cid, pl.ds(sid * NPT, NPT)])

        @pl.when(sid == NS - 1)
        def _():
            pltpu.sync_copy(acc.at[pl.ds(NS * NPT, NREM)],
                            out_hbm.at[cid, pl.ds(NS * NPT, NREM)])

    return k(vals0, vals1, row, zeros)


BE = 2000


def _edge_specs(sl):
    off = sl * (ES // BE)
    return [
        pl.BlockSpec((BE, H), lambda i: (i, 0)),
        pl.BlockSpec((BE, 1), lambda i: (i + off, 0)),
        pl.BlockSpec((BE, 1), lambda i: (i + off, 0)),
        pl.BlockSpec((H, H), lambda i: (0, 0)),
        pl.BlockSpec((8, H), lambda i: (0, 0)),
    ]


def _gcl_edge_body(g, rad, ea, w2t, aux, out):
    radial = rad[...]
    pre = (g[...].astype(f32)
           + radial * aux[0:1, :]
           + ea[...] * aux[1:2, :])
    m1 = pre * jax.nn.sigmoid(pre)
    z = jnp.dot(m1, w2t[...], preferred_element_type=f32) + aux[2:3, :]
    m2 = z * jax.nn.sigmoid(z)
    att = jax.nn.sigmoid(
        jnp.sum(m2 * aux[3:4, :], axis=1, keepdims=True) + aux[4, 0])
    out[...] = m2 * att


def _gcl_edge(g, rad, ea, w2t, aux, sl):
    return pl.pallas_call(
        _gcl_edge_body,
        grid=(ES // BE,),
        in_specs=_edge_specs(sl),
        out_specs=pl.BlockSpec((BE, H), lambda i: (i, 0)),
        out_shape=jax.ShapeDtypeStruct((ES, H), f32),
    )(g, rad, ea, w2t, aux)


def _eq_edge_body(g, rad, ea, w2t, aux, out):
    radial = rad[...]
    pre = (g[...].astype(f32)
           + radial * aux[0:1, :]
           + ea[...] * aux[1:2, :])
    t1 = pre * jax.nn.sigmoid(pre)
    z = jnp.dot(t1, w2t[...], preferred_element_type=f32) + aux[2:3, :]
    t2 = z * jax.nn.sigmoid(z)
    phi = jnp.sum(t2 * aux[3:4, :], axis=1, keepdims=True)
    out[...] = phi / (jnp.sqrt(radial + 1e-8) + NORM_CONST)


def _eq_edge(g, rad, ea, w2t, aux, sl):
    return pl.pallas_call(
        _eq_edge_body,
        grid=(ES // BE,),
        in_specs=_edge_specs(sl),
        out_specs=pl.BlockSpec((BE, 1), lambda i: (i, 0)),
        out_shape=jax.ShapeDtypeStruct((ES, 1), f32),
    )(g, rad, ea, w2t, aux)


BN = 2000


def _proj_body(h, wrt, wct, aux, a_out, b_out):
    hv = h[...]
    a_out[...] = jnp.dot(hv, wrt[...], preferred_element_type=f32) + aux[0:1, :]
    b_out[...] = jnp.dot(hv, wct[...], preferred_element_type=f32)


def _proj2(h, wrt, wct, aux):
    return pl.pallas_call(
        _proj_body,
        grid=(N // BN,),
        in_specs=[
            pl.BlockSpec((BN, H), lambda i: (i, 0)),
            pl.BlockSpec((H, H), lambda i: (0, 0)),
            pl.BlockSpec((H, H), lambda i: (0, 0)),
            pl.BlockSpec((8, H), lambda i: (0, 0)),
        ],
        out_specs=[pl.BlockSpec((BN, H), lambda i: (i, 0)),
                   pl.BlockSpec((BN, H), lambda i: (i, 0))],
        out_shape=[jax.ShapeDtypeStruct((N, H), f32),
                   jax.ShapeDtypeStruct((N, H), f32)],
    )(h, wrt, wct, aux)


def _node_body(h, p, w1ht, w1at, w2t, wrnt, wcnt, aux,
               hn_out, an_out, bn_out):
    hv = h[...]
    agg = (p[0] + p[1]) * (1.0 / NORM_FACTOR)
    z = (jnp.dot(hv, w1ht[...], preferred_element_type=f32)
         + jnp.dot(agg, w1at[...], preferred_element_type=f32)
         + aux[0:1, :])
    t = z * jax.nn.sigmoid(z)
    hn = hv + jnp.dot(t, w2t[...], preferred_element_type=f32) + aux[1:2, :]
    hn_out[...] = hn
    an_out[...] = jnp.dot(hn, wrnt[...], preferred_element_type=f32) + aux[2:3, :]
    bn_out[...] = jnp.dot(hn, wcnt[...], preferred_element_type=f32)


def _node(h, p, w1ht, w1at, w2t, wrnt, wcnt, aux):
    wspec = pl.BlockSpec((H, H), lambda i: (0, 0))
    pspec = pl.BlockSpec((NC, BN, H), lambda i: (0, i, 0))
    return pl.pallas_call(
        _node_body,
        grid=(N // BN,),
        in_specs=[
            pl.BlockSpec((BN, H), lambda i: (i, 0)),
            pspec,
            wspec, wspec, wspec, wspec, wspec,
            pl.BlockSpec((8, H), lambda i: (0, 0)),
        ],
        out_specs=[pl.BlockSpec((BN, H), lambda i: (i, 0))] * 3,
        out_shape=[jax.ShapeDtypeStruct((N, H), f32)] * 3,
    )(h, p, w1ht, w1at, w2t, wrnt, wcnt, aux)


def _phase_weights(W1, b1):
    wrt = W1[:, :H].T
    wct = W1[:, H:2 * H].T
    wr = W1[:, 2 * H]
    wa = W1[:, 2 * H + 1]
    return wrt, wct, wr, wa, b1


def _pack8(*rows):
    aux = jnp.zeros((8, H), f32)
    for i, r in enumerate(rows):
        aux = aux.at[i, :r.shape[0]].set(r)
    return aux


def kernel(h, x, edge_attr, params, edge_index):
    row = edge_index[0].astype(i32)
    col = edge_index[1].astype(i32)
    ea = edge_attr.astype(f32)

    xs0, xs1, xs2 = x[:, 0], x[:, 1], x[:, 2]
    radial = _radial_kernel(xs0, xs1, xs2, row, col).reshape(E, 1)

    zeros128 = jnp.zeros((N, H), f32)
    zeros_n = jnp.zeros((N,), f32)

    g0, g1 = params["gcl"][0], params["gcl"][1]
    q = params["eq"]
    wrt0, wct0, wr0, wa0, b10 = _phase_weights(g0["e_w1"], g0["e_b1"])
    wrt1, wct1, wr1, wa1, b11 = _phase_weights(g1["e_w1"], g1["e_b1"])
    wrtq, wctq, wrq, waq, b1q = _phase_weights(q["w1"], q["b1"])

    A, B = _proj2(h, wrt0, wct0, _pack8(b10))

    for l, (p, wr, wa, nxt) in enumerate([
        (g0, wr0, wa0, (wrt1, wct1, b11)),
        (g1, wr1, wa1, (wrtq, wctq, b1q)),
    ]):
        aux_e = _pack8(wr, wa, p["e_b2"], p["a_w"][0], p["a_b"])
        w2t = p["e_w2"].T
        efs = []
        for sl in range(NSLICE):
            G = _gather_add(A, B, row, col, sl)
            efs.append(_gcl_edge(G, radial, ea, w2t, aux_e, sl))
        P = _scatter_add(efs[0], efs[1], row, zeros128)
        h, A, B = _node(
            h, P,
            p["n_w1"][:, :H].T, p["n_w1"][:, H:].T, p["n_w2"].T,
            nxt[0], nxt[1], _pack8(p["n_b1"], p["n_b2"], nxt[2]))

    aux_q = _pack8(wrq, waq, q["b2"], q["w3"][0])
    w2tq = q["w2"].T
    ss = []
    for sl in range(NSLICE):
        G = _gather_add(A, B, row, col, sl)
        ss.append(_eq_edge(G, radial, ea, w2tq, aux_q, sl))
    s = jnp.concatenate(ss, axis=0)
    p0, p1, p2 = _eq_scatter(s.reshape(E), row, col, xs0, xs1, xs2, zeros_n)
    aggx = jnp.stack(
        [p0[:N] + p0[N:], p1[:N] + p1[N:], p2[:N] + p2[N:]], axis=1)
    x_out = x + aggx * (1.0 / NORM_FACTOR)
    return h, x_out


# speedup vs baseline: 1.0783x; 1.0783x over previous
import functools

import jax
import jax.numpy as jnp
from jax import lax
from jax.experimental import pallas as pl
from jax.experimental.pallas import tpu as pltpu
from jax.experimental.pallas import tpu_sc as plsc

N = 10000
E = 320000
H = 128
NORM_FACTOR = 100.0
NORM_CONST = 1.0

NC = 2
NS = 16
NW = NC * NS
EPW = E // NW
NSLICE = 2
ES = E // NSLICE
ESW = ES // NW
CHUNK = 400
NPT = 624
NREM = N - NS * NPT

f32 = jnp.float32
bf16 = jnp.bfloat16
i32 = jnp.int32


def _mesh():
    return plsc.VectorSubcoreMesh(core_axis_name="c", subcore_axis_name="s")


def _gather_add(A, B, row, col, sl):
    C = 200
    NCHUNK = ESW // C
    NPAIR = (NCHUNK - 1) // 2

    @functools.partial(
        pl.kernel,
        mesh=_mesh(),
        compiler_params=pltpu.CompilerParams(needs_layout_passes=False),
        out_type=jax.ShapeDtypeStruct((ES, H), f32),
        scratch_types=[
            pltpu.VMEM((ESW,), i32), pltpu.VMEM((ESW,), i32),
            pltpu.VMEM((C, H), f32), pltpu.VMEM((C, H), f32),
            pltpu.VMEM((C, H), f32), pltpu.VMEM((C, H), f32),
            pltpu.SemaphoreType.DMA, pltpu.SemaphoreType.DMA,
        ],
    )
    def k(a_hbm, b_hbm, row_hbm, col_hbm, g_hbm,
          rv, cv, a0, b0, a1, b1, gsem, wsem):
        wid = lax.axis_index("s") * NC + lax.axis_index("c")
        base = wid * ESW
        gbase = sl * ES + base
        pltpu.sync_copy(row_hbm.at[pl.ds(gbase, ESW)], rv)
        pltpu.sync_copy(col_hbm.at[pl.ds(gbase, ESW)], cv)

        def addloop(abuf, bbuf):
            def arow(r, c2):
                for j in range(H // 16):
                    slc = pl.ds(j * 16, 16)
                    abuf[r, slc] = abuf[r, slc] + bbuf[r, slc]
                return c2

            lax.fori_loop(0, C, arow, 0)

        def body(i, carry):
            loc0 = (2 * i) * C
            loc1 = loc0 + C
            off0 = base + loc0
            off1 = base + loc1

            @pl.when(i > 0)
            def _():
                pltpu.make_async_copy(a0, g_hbm.at[pl.ds(off0, C)], wsem).wait()
                pltpu.make_async_copy(a1, g_hbm.at[pl.ds(off1, C)], wsem).wait()

            g0a = pltpu.async_copy(a_hbm.at[rv.at[pl.ds(loc0, C)]], a0, gsem)
            g0b = pltpu.async_copy(b_hbm.at[cv.at[pl.ds(loc0, C)]], b0, gsem)
            g0a.wait()
            g0b.wait()
            g1a = pltpu.async_copy(a_hbm.at[rv.at[pl.ds(loc1, C)]], a1, gsem)
            g1b = pltpu.async_copy(b_hbm.at[cv.at[pl.ds(loc1, C)]], b1, gsem)
            addloop(a0, b0)
            pltpu.async_copy(a0, g_hbm.at[pl.ds(off0, C)], wsem)
            g1a.wait()
            g1b.wait()
            addloop(a1, b1)
            pltpu.async_copy(a1, g_hbm.at[pl.ds(off1, C)], wsem)
            return carry

        lax.fori_loop(0, NPAIR, body, 0)
        tloc = (2 * NPAIR) * C
        pltpu.make_async_copy(a0, g_hbm.at[pl.ds(base, C)], wsem).wait()
        ta = pltpu.async_copy(a_hbm.at[rv.at[pl.ds(tloc, C)]], a0, gsem)
        tb = pltpu.async_copy(b_hbm.at[cv.at[pl.ds(tloc, C)]], b0, gsem)
        ta.wait()
        tb.wait()
        addloop(a0, b0)
        pltpu.sync_copy(a0, g_hbm.at[pl.ds(base + tloc, C)])
        pltpu.make_async_copy(a1, g_hbm.at[pl.ds(base, C)], wsem).wait()

    return k(A, B, row, col)


def _radial_kernel(xs0, xs1, xs2, row, col):
    @functools.partial(
        pl.kernel,
        mesh=_mesh(),
        compiler_params=pltpu.CompilerParams(needs_layout_passes=False),
        out_type=jax.ShapeDtypeStruct((E,), f32),
        scratch_types=[
            pltpu.VMEM((N,), f32),
            pltpu.VMEM((N,), f32),
            pltpu.VMEM((N,), f32),
            pltpu.VMEM((CHUNK,), i32),
            pltpu.VMEM((CHUNK,), i32),
            pltpu.VMEM((CHUNK,), f32),
        ],
    )
    def k(x0_hbm, x1_hbm, x2_hbm, row_hbm, col_hbm, r_hbm,
          t0, t1, t2, rv, cv, rbuf):
        wid = lax.axis_index("s") * NC + lax.axis_index("c")
        pltpu.sync_copy(x0_hbm, t0)
        pltpu.sync_copy(x1_hbm, t1)
        pltpu.sync_copy(x2_hbm, t2)
        base = wid * EPW

        def body(i, carry):
            off = base + i * CHUNK
            pltpu.sync_copy(row_hbm.at[pl.ds(off, CHUNK)], rv)
            pltpu.sync_copy(col_hbm.at[pl.ds(off, CHUNK)], cv)

            def grp(j, c2):
                ridx = rv[pl.ds(j * 16, 16)]
                cidx = cv[pl.ds(j * 16, 16)]
                d0 = plsc.load_gather(t0, [ridx]) - plsc.load_gather(t0, [cidx])
                d1 = plsc.load_gather(t1, [ridx]) - plsc.load_gather(t1, [cidx])
                d2 = plsc.load_gather(t2, [ridx]) - plsc.load_gather(t2, [cidx])
                rbuf[pl.ds(j * 16, 16)] = d0 * d0 + d1 * d1 + d2 * d2
                return c2

            lax.fori_loop(0, CHUNK // 16, grp, 0)
            pltpu.sync_copy(rbuf, r_hbm.at[pl.ds(off, CHUNK)])
            return carry

        lax.fori_loop(0, EPW // CHUNK, body, 0)

    return k(xs0, xs1, xs2, row, col)


def _eq_scatter(s, row, col, xs0, xs1, xs2, zeros_n):
    @functools.partial(
        pl.kernel,
        mesh=_mesh(),
        compiler_params=pltpu.CompilerParams(needs_layout_passes=False),
        out_type=[jax.ShapeDtypeStruct((NC * N,), f32)] * 3,
        scratch_types=[
            pltpu.VMEM((N,), f32),
            pltpu.VMEM((N,), f32),
            pltpu.VMEM((N,), f32),
            pltpu.VMEM((CHUNK,), i32),
            pltpu.VMEM((CHUNK,), i32),
            pltpu.VMEM((CHUNK,), f32),
            pltpu.VMEM((CHUNK,), f32),
            pltpu.VMEM((CHUNK,), f32),
            pltpu.VMEM((CHUNK,), f32),
            pltpu.VMEM((N,), f32),
            pltpu.VMEM_SHARED((N,), f32),
            pltpu.VMEM_SHARED((N,), f32),
            pltpu.VMEM_SHARED((N,), f32),
        ],
    )
    def k(s_hbm, row_hbm, col_hbm, x0_hbm, x1_hbm, x2_hbm, z_hbm,
          o0_hbm, o1_hbm, o2_hbm,
          t0, t1, t2, rv, cv, sv, f0, f1, f2, stage, acc0, acc1, acc2):
        cid = lax.axis_index("c")
        sid = lax.axis_index("s")
        wid = sid * NC + cid
        pltpu.sync_copy(x0_hbm, t0)
        pltpu.sync_copy(x1_hbm, t1)
        pltpu.sync_copy(x2_hbm, t2)

        accs = [acc0, acc1, acc2]
        for kk in range(3):
            @pl.when(sid == kk)
            def _(kk=kk):
                pltpu.sync_copy(z_hbm, stage)
                pltpu.sync_copy(stage, accs[kk])

        plsc.subcore_barrier()
        base = wid * EPW

        def body(i, carry):
            off = base + i * CHUNK
            pltpu.sync_copy(row_hbm.at[pl.ds(off, CHUNK)], rv)
            pltpu.sync_copy(col_hbm.at[pl.ds(off, CHUNK)], cv)
            pltpu.sync_copy(s_hbm.at[pl.ds(off, CHUNK)], sv)

            def grp(j, c2):
                sl = pl.ds(j * 16, 16)
                ridx = rv[sl]
                cidx = cv[sl]
                se = sv[sl]
                f0[sl] = (plsc.load_gather(t0, [ridx])
                          - plsc.load_gather(t0, [cidx])) * se
                f1[sl] = (plsc.load_gather(t1, [ridx])
                          - plsc.load_gather(t1, [cidx])) * se
                f2[sl] = (plsc.load_gather(t2, [ridx])
                          - plsc.load_gather(t2, [cidx])) * se
                return c2

            lax.fori_loop(0, CHUNK // 16, grp, 0)
            pltpu.sync_copy(f0, acc0.at[rv], add=True)
            pltpu.sync_copy(f1, acc1.at[rv], add=True)
            pltpu.sync_copy(f2, acc2.at[rv], add=True)
            return carry

        lax.fori_loop(0, EPW // CHUNK, body, 0)
        plsc.subcore_barrier()

        outs = [o0_hbm, o1_hbm, o2_hbm]
        for kk in range(3):
            @pl.when(sid == kk)
            def _(kk=kk):
                pltpu.sync_copy(accs[kk], stage)
                pltpu.sync_copy(stage, outs[kk].at[pl.ds(cid * N, N)])

    return k(s, row, col, xs0, xs1, xs2, zeros_n)


def _scatter_add(vals, row, zeros, sl, D=H, CS=40):
    NPAIR = (ESW // CS - 1) // 2

    @functools.partial(
        pl.kernel,
        mesh=_mesh(),
        out_type=jax.ShapeDtypeStruct((NC, N, D), f32),
        scratch_types=[
            pltpu.VMEM((CS,), i32),
            pltpu.VMEM((CS,), i32),
            pltpu.VMEM((CS, D), f32),
            pltpu.VMEM((CS, D), f32),
            pltpu.VMEM_SHARED((N, D), f32),
            pltpu.SemaphoreType.DMA,
        ],
    )
    def k(vals_hbm, row_hbm, zeros_hbm, out_hbm, i0, i1, v0, v1, acc, sem):
        cid = lax.axis_index("c")
        sid = lax.axis_index("s")
        wid = sid * NC + cid
        pltpu.sync_copy(zeros_hbm.at[pl.ds(sid * NPT, NPT)],
                        acc.at[pl.ds(sid * NPT, NPT)])

        @pl.when(sid == NS - 1)
        def _():
            pltpu.sync_copy(zeros_hbm.at[pl.ds(NS * NPT, NREM)],
                            acc.at[pl.ds(NS * NPT, NREM)])

        plsc.subcore_barrier()
        base = wid * ESW
        gbase = sl * ES + base
        pltpu.sync_copy(row_hbm.at[pl.ds(gbase, CS)], i0)
        pltpu.sync_copy(vals_hbm.at[pl.ds(base, CS)], v0)

        def body(i, carry):
            c1 = (2 * i + 1) * CS
            c2 = c1 + CS
            l1a = pltpu.async_copy(row_hbm.at[pl.ds(gbase + c1, CS)], i1, sem)
            l1b = pltpu.async_copy(vals_hbm.at[pl.ds(base + c1, CS)], v1, sem)
            pltpu.sync_copy(v0, acc.at[i0], add=True)
            l1a.wait()
            l1b.wait()
            l0a = pltpu.async_copy(row_hbm.at[pl.ds(gbase + c2, CS)], i0, sem)
            l0b = pltpu.async_copy(vals_hbm.at[pl.ds(base + c2, CS)], v0, sem)
            pltpu.sync_copy(v1, acc.at[i1], add=True)
            l0a.wait()
            l0b.wait()
            return carry

        lax.fori_loop(0, NPAIR, body, 0)
        pltpu.sync_copy(v0, acc.at[i0], add=True)
        plsc.subcore_barrier()
        pltpu.sync_copy(acc.at[pl.ds(sid * NPT, NPT)],
                        out_hbm.at[cid, pl.ds(sid * NPT, NPT)])

        @pl.when(sid == NS - 1)
        def _():
            pltpu.sync_copy(acc.at[pl.ds(NS * NPT, NREM)],
                            out_hbm.at[cid, pl.ds(NS * NPT, NREM)])

    return k(vals, row, zeros)


BE = 2000


def _edge_specs(sl):
    off = sl * (ES // BE)
    return [
        pl.BlockSpec((BE, H), lambda i: (i, 0)),
        pl.BlockSpec((BE, 1), lambda i: (i + off, 0)),
        pl.BlockSpec((BE, 1), lambda i: (i + off, 0)),
        pl.BlockSpec((H, H), lambda i: (0, 0)),
        pl.BlockSpec((8, H), lambda i: (0, 0)),
    ]


def _gcl_edge_body(g, rad, ea, w2t, aux, out):
    radial = rad[...]
    pre = (g[...].astype(f32)
           + radial * aux[0:1, :]
           + ea[...] * aux[1:2, :])
    m1 = pre * jax.nn.sigmoid(pre)
    z = jnp.dot(m1, w2t[...], preferred_element_type=f32) + aux[2:3, :]
    m2 = z * jax.nn.sigmoid(z)
    att = jax.nn.sigmoid(
        jnp.sum(m2 * aux[3:4, :], axis=1, keepdims=True) + aux[4, 0])
    out[...] = m2 * att


def _gcl_edge(g, rad, ea, w2t, aux, sl):
    return pl.pallas_call(
        _gcl_edge_body,
        grid=(ES // BE,),
        in_specs=_edge_specs(sl),
        out_specs=pl.BlockSpec((BE, H), lambda i: (i, 0)),
        out_shape=jax.ShapeDtypeStruct((ES, H), f32),
    )(g, rad, ea, w2t, aux)


def _eq_edge_body(g, rad, ea, w2t, aux, out):
    radial = rad[...]
    pre = (g[...].astype(f32)
           + radial * aux[0:1, :]
           + ea[...] * aux[1:2, :])
    t1 = pre * jax.nn.sigmoid(pre)
    z = jnp.dot(t1, w2t[...], preferred_element_type=f32) + aux[2:3, :]
    t2 = z * jax.nn.sigmoid(z)
    phi = jnp.sum(t2 * aux[3:4, :], axis=1, keepdims=True)
    out[...] = phi / (jnp.sqrt(radial + 1e-8) + NORM_CONST)


def _eq_edge(g, rad, ea, w2t, aux, sl):
    return pl.pallas_call(
        _eq_edge_body,
        grid=(ES // BE,),
        in_specs=_edge_specs(sl),
        out_specs=pl.BlockSpec((BE, 1), lambda i: (i, 0)),
        out_shape=jax.ShapeDtypeStruct((ES, 1), f32),
    )(g, rad, ea, w2t, aux)


BN = 2000


def _proj_body(h, wrt, wct, aux, a_out, b_out):
    hv = h[...]
    a_out[...] = jnp.dot(hv, wrt[...], preferred_element_type=f32) + aux[0:1, :]
    b_out[...] = jnp.dot(hv, wct[...], preferred_element_type=f32)


def _proj2(h, wrt, wct, aux):
    return pl.pallas_call(
        _proj_body,
        grid=(N // BN,),
        in_specs=[
            pl.BlockSpec((BN, H), lambda i: (i, 0)),
            pl.BlockSpec((H, H), lambda i: (0, 0)),
            pl.BlockSpec((H, H), lambda i: (0, 0)),
            pl.BlockSpec((8, H), lambda i: (0, 0)),
        ],
        out_specs=[pl.BlockSpec((BN, H), lambda i: (i, 0)),
                   pl.BlockSpec((BN, H), lambda i: (i, 0))],
        out_shape=[jax.ShapeDtypeStruct((N, H), f32),
                   jax.ShapeDtypeStruct((N, H), f32)],
    )(h, wrt, wct, aux)


def _node_body(h, p0, p1, w1ht, w1at, w2t, wrnt, wcnt, aux,
               hn_out, an_out, bn_out):
    hv = h[...]
    agg = (p0[0] + p0[1] + p1[0] + p1[1]) * (1.0 / NORM_FACTOR)
    z = (jnp.dot(hv, w1ht[...], preferred_element_type=f32)
         + jnp.dot(agg, w1at[...], preferred_element_type=f32)
         + aux[0:1, :])
    t = z * jax.nn.sigmoid(z)
    hn = hv + jnp.dot(t, w2t[...], preferred_element_type=f32) + aux[1:2, :]
    hn_out[...] = hn
    an_out[...] = jnp.dot(hn, wrnt[...], preferred_element_type=f32) + aux[2:3, :]
    bn_out[...] = jnp.dot(hn, wcnt[...], preferred_element_type=f32)


def _node(h, p0, p1, w1ht, w1at, w2t, wrnt, wcnt, aux):
    wspec = pl.BlockSpec((H, H), lambda i: (0, 0))
    pspec = pl.BlockSpec((NC, BN, H), lambda i: (0, i, 0))
    return pl.pallas_call(
        _node_body,
        grid=(N // BN,),
        in_specs=[
            pl.BlockSpec((BN, H), lambda i: (i, 0)),
            pspec, pspec,
            wspec, wspec, wspec, wspec, wspec,
            pl.BlockSpec((8, H), lambda i: (0, 0)),
        ],
        out_specs=[pl.BlockSpec((BN, H), lambda i: (i, 0))] * 3,
        out_shape=[jax.ShapeDtypeStruct((N, H), f32)] * 3,
    )(h, p0, p1, w1ht, w1at, w2t, wrnt, wcnt, aux)


def _phase_weights(W1, b1):
    wrt = W1[:, :H].T
    wct = W1[:, H:2 * H].T
    wr = W1[:, 2 * H]
    wa = W1[:, 2 * H + 1]
    return wrt, wct, wr, wa, b1


def _pack8(*rows):
    aux = jnp.zeros((8, H), f32)
    for i, r in enumerate(rows):
        aux = aux.at[i, :r.shape[0]].set(r)
    return aux


def kernel(h, x, edge_attr, params, edge_index):
    row = edge_index[0].astype(i32)
    col = edge_index[1].astype(i32)
    ea = edge_attr.astype(f32)

    xs0, xs1, xs2 = x[:, 0], x[:, 1], x[:, 2]
    radial = _radial_kernel(xs0, xs1, xs2, row, col).reshape(E, 1)

    zeros128 = jnp.zeros((N, H), f32)
    zeros_n = jnp.zeros((N,), f32)

    g0, g1 = params["gcl"][0], params["gcl"][1]
    q = params["eq"]
    wrt0, wct0, wr0, wa0, b10 = _phase_weights(g0["e_w1"], g0["e_b1"])
    wrt1, wct1, wr1, wa1, b11 = _phase_weights(g1["e_w1"], g1["e_b1"])
    wrtq, wctq, wrq, waq, b1q = _phase_weights(q["w1"], q["b1"])

    A, B = _proj2(h, wrt0, wct0, _pack8(b10))

    for l, (p, wr, wa, nxt) in enumerate([
        (g0, wr0, wa0, (wrt1, wct1, b11)),
        (g1, wr1, wa1, (wrtq, wctq, b1q)),
    ]):
        aux_e = _pack8(wr, wa, p["e_b2"], p["a_w"][0], p["a_b"])
        w2t = p["e_w2"].T
        Ps = []
        for sl in range(NSLICE):
            G = _gather_add(A, B, row, col, sl)
            ef = _gcl_edge(G, radial, ea, w2t, aux_e, sl)
            Ps.append(_scatter_add(ef, row, zeros128, sl))
        h, A, B = _node(
            h, Ps[0], Ps[1],
            p["n_w1"][:, :H].T, p["n_w1"][:, H:].T, p["n_w2"].T,
            nxt[0], nxt[1], _pack8(p["n_b1"], p["n_b2"], nxt[2]))

    aux_q = _pack8(wrq, waq, q["b2"], q["w3"][0])
    w2tq = q["w2"].T
    ss = []
    for sl in range(NSLICE):
        G = _gather_add(A, B, row, col, sl)
        ss.append(_eq_edge(G, radial, ea, w2tq, aux_q, sl))
    s = jnp.concatenate(ss, axis=0)
    p0, p1, p2 = _eq_scatter(s.reshape(E), row, col, xs0, xs1, xs2, zeros_n)
    aggx = jnp.stack(
        [p0[:N] + p0[N:], p1[:N] + p1[N:], p2[:N] + p2[N:]], axis=1)
    x_out = x + aggx * (1.0 / NORM_FACTOR)
    return h, x_out


# speedup vs baseline: 1.1040x; 1.0239x over previous
import functools

import jax
import jax.numpy as jnp
from jax import lax
from jax.experimental import pallas as pl
from jax.experimental.pallas import tpu as pltpu
from jax.experimental.pallas import tpu_sc as plsc

N = 10000
E = 320000
H = 128
NORM_FACTOR = 100.0
NORM_CONST = 1.0

NC = 2
NS = 16
NW = NC * NS
EPW = E // NW
NSLICE = 2
ES = E // NSLICE
ESW = ES // NW
CHUNK = 400
NPT = 624
NREM = N - NS * NPT

f32 = jnp.float32
bf16 = jnp.bfloat16
i32 = jnp.int32


def _mesh():
    return plsc.VectorSubcoreMesh(core_axis_name="c", subcore_axis_name="s")


def _gather_add(A, B, row, col, sl):
    C = 200
    NCHUNK = ESW // C
    NPAIR = (NCHUNK - 1) // 2

    @functools.partial(
        pl.kernel,
        mesh=_mesh(),
        compiler_params=pltpu.CompilerParams(needs_layout_passes=False),
        out_type=jax.ShapeDtypeStruct((ES, H), f32),
        scratch_types=[
            pltpu.VMEM((ESW,), i32), pltpu.VMEM((ESW,), i32),
            pltpu.VMEM((C, H), f32), pltpu.VMEM((C, H), f32),
            pltpu.VMEM((C, H), f32), pltpu.VMEM((C, H), f32),
            pltpu.SemaphoreType.DMA, pltpu.SemaphoreType.DMA,
        ],
    )
    def k(a_hbm, b_hbm, row_hbm, col_hbm, g_hbm,
          rv, cv, a0, b0, a1, b1, gsem, wsem):
        wid = lax.axis_index("s") * NC + lax.axis_index("c")
        base = wid * ESW
        gbase = sl * ES + base
        pltpu.sync_copy(row_hbm.at[pl.ds(gbase, ESW)], rv)
        pltpu.sync_copy(col_hbm.at[pl.ds(gbase, ESW)], cv)

        def addloop(abuf, bbuf):
            def arow(r, c2):
                for j in range(H // 16):
                    slc = pl.ds(j * 16, 16)
                    abuf[r, slc] = abuf[r, slc] + bbuf[r, slc]
                return c2

            lax.fori_loop(0, C, arow, 0)

        def body(i, carry):
            loc0 = (2 * i) * C
            loc1 = loc0 + C
            off0 = base + loc0
            off1 = base + loc1

            @pl.when(i > 0)
            def _():
                pltpu.make_async_copy(a0, g_hbm.at[pl.ds(off0, C)], wsem).wait()
                pltpu.make_async_copy(a1, g_hbm.at[pl.ds(off1, C)], wsem).wait()

            g0a = pltpu.async_copy(a_hbm.at[rv.at[pl.ds(loc0, C)]], a0, gsem)
            g0b = pltpu.async_copy(b_hbm.at[cv.at[pl.ds(loc0, C)]], b0, gsem)
            g0a.wait()
            g0b.wait()
            g1a = pltpu.async_copy(a_hbm.at[rv.at[pl.ds(loc1, C)]], a1, gsem)
            g1b = pltpu.async_copy(b_hbm.at[cv.at[pl.ds(loc1, C)]], b1, gsem)
            addloop(a0, b0)
            pltpu.async_copy(a0, g_hbm.at[pl.ds(off0, C)], wsem)
            g1a.wait()
            g1b.wait()
            addloop(a1, b1)
            pltpu.async_copy(a1, g_hbm.at[pl.ds(off1, C)], wsem)
            return carry

        lax.fori_loop(0, NPAIR, body, 0)
        tloc = (2 * NPAIR) * C
        pltpu.make_async_copy(a0, g_hbm.at[pl.ds(base, C)], wsem).wait()
        ta = pltpu.async_copy(a_hbm.at[rv.at[pl.ds(tloc, C)]], a0, gsem)
        tb = pltpu.async_copy(b_hbm.at[cv.at[pl.ds(tloc, C)]], b0, gsem)
        ta.wait()
        tb.wait()
        addloop(a0, b0)
        pltpu.sync_copy(a0, g_hbm.at[pl.ds(base + tloc, C)])
        pltpu.make_async_copy(a1, g_hbm.at[pl.ds(base, C)], wsem).wait()

    return k(A, B, row, col)


def _radial_kernel(xs0, xs1, xs2, row, col):
    @functools.partial(
        pl.kernel,
        mesh=_mesh(),
        compiler_params=pltpu.CompilerParams(needs_layout_passes=False),
        out_type=jax.ShapeDtypeStruct((E,), f32),
        scratch_types=[
            pltpu.VMEM((N,), f32),
            pltpu.VMEM((N,), f32),
            pltpu.VMEM((N,), f32),
            pltpu.VMEM((CHUNK,), i32),
            pltpu.VMEM((CHUNK,), i32),
            pltpu.VMEM((CHUNK,), f32),
        ],
    )
    def k(x0_hbm, x1_hbm, x2_hbm, row_hbm, col_hbm, r_hbm,
          t0, t1, t2, rv, cv, rbuf):
        wid = lax.axis_index("s") * NC + lax.axis_index("c")
        pltpu.sync_copy(x0_hbm, t0)
        pltpu.sync_copy(x1_hbm, t1)
        pltpu.sync_copy(x2_hbm, t2)
        base = wid * EPW

        def body(i, carry):
            off = base + i * CHUNK
            pltpu.sync_copy(row_hbm.at[pl.ds(off, CHUNK)], rv)
            pltpu.sync_copy(col_hbm.at[pl.ds(off, CHUNK)], cv)

            def grp(j, c2):
                ridx = rv[pl.ds(j * 16, 16)]
                cidx = cv[pl.ds(j * 16, 16)]
                d0 = plsc.load_gather(t0, [ridx]) - plsc.load_gather(t0, [cidx])
                d1 = plsc.load_gather(t1, [ridx]) - plsc.load_gather(t1, [cidx])
                d2 = plsc.load_gather(t2, [ridx]) - plsc.load_gather(t2, [cidx])
                rbuf[pl.ds(j * 16, 16)] = d0 * d0 + d1 * d1 + d2 * d2
                return c2

            lax.fori_loop(0, CHUNK // 16, grp, 0)
            pltpu.sync_copy(rbuf, r_hbm.at[pl.ds(off, CHUNK)])
            return carry

        lax.fori_loop(0, EPW // CHUNK, body, 0)

    return k(xs0, xs1, xs2, row, col)


def _eq_scatter(s, row, col, xs0, xs1, xs2, zeros_n):
    @functools.partial(
        pl.kernel,
        mesh=_mesh(),
        compiler_params=pltpu.CompilerParams(needs_layout_passes=False),
        out_type=[jax.ShapeDtypeStruct((NC * N,), f32)] * 3,
        scratch_types=[
            pltpu.VMEM((N,), f32),
            pltpu.VMEM((N,), f32),
            pltpu.VMEM((N,), f32),
            pltpu.VMEM((CHUNK,), i32),
            pltpu.VMEM((CHUNK,), i32),
            pltpu.VMEM((CHUNK,), f32),
            pltpu.VMEM((CHUNK,), f32),
            pltpu.VMEM((CHUNK,), f32),
            pltpu.VMEM((CHUNK,), f32),
            pltpu.VMEM((N,), f32),
            pltpu.VMEM_SHARED((N,), f32),
            pltpu.VMEM_SHARED((N,), f32),
            pltpu.VMEM_SHARED((N,), f32),
        ],
    )
    def k(s_hbm, row_hbm, col_hbm, x0_hbm, x1_hbm, x2_hbm, z_hbm,
          o0_hbm, o1_hbm, o2_hbm,
          t0, t1, t2, rv, cv, sv, f0, f1, f2, stage, acc0, acc1, acc2):
        cid = lax.axis_index("c")
        sid = lax.axis_index("s")
        wid = sid * NC + cid
        pltpu.sync_copy(x0_hbm, t0)
        pltpu.sync_copy(x1_hbm, t1)
        pltpu.sync_copy(x2_hbm, t2)

        accs = [acc0, acc1, acc2]
        for kk in range(3):
            @pl.when(sid == kk)
            def _(kk=kk):
                pltpu.sync_copy(z_hbm, stage)
                pltpu.sync_copy(stage, accs[kk])

        plsc.subcore_barrier()
        base = wid * EPW

        def body(i, carry):
            off = base + i * CHUNK
            pltpu.sync_copy(row_hbm.at[pl.ds(off, CHUNK)], rv)
            pltpu.sync_copy(col_hbm.at[pl.ds(off, CHUNK)], cv)
            pltpu.sync_copy(s_hbm.at[pl.ds(off, CHUNK)], sv)

            def grp(j, c2):
                sl = pl.ds(j * 16, 16)
                ridx = rv[sl]
                cidx = cv[sl]
                se = sv[sl]
                f0[sl] = (plsc.load_gather(t0, [ridx])
                          - plsc.load_gather(t0, [cidx])) * se
                f1[sl] = (plsc.load_gather(t1, [ridx])
                          - plsc.load_gather(t1, [cidx])) * se
                f2[sl] = (plsc.load_gather(t2, [ridx])
                          - plsc.load_gather(t2, [cidx])) * se
                return c2

            lax.fori_loop(0, CHUNK // 16, grp, 0)
            pltpu.sync_copy(f0, acc0.at[rv], add=True)
            pltpu.sync_copy(f1, acc1.at[rv], add=True)
            pltpu.sync_copy(f2, acc2.at[rv], add=True)
            return carry

        lax.fori_loop(0, EPW // CHUNK, body, 0)
        plsc.subcore_barrier()

        outs = [o0_hbm, o1_hbm, o2_hbm]
        for kk in range(3):
            @pl.when(sid == kk)
            def _(kk=kk):
                pltpu.sync_copy(accs[kk], stage)
                pltpu.sync_copy(stage, outs[kk].at[pl.ds(cid * N, N)])

    return k(s, row, col, xs0, xs1, xs2, zeros_n)


def _scatter_add(vals, row, zeros, sl, D=H, CS=40):
    NPAIR = (ESW // CS - 1) // 2

    @functools.partial(
        pl.kernel,
        mesh=_mesh(),
        out_type=jax.ShapeDtypeStruct((NC, N, D), f32),
        scratch_types=[
            pltpu.VMEM((CS,), i32),
            pltpu.VMEM((CS,), i32),
            pltpu.VMEM((CS, D), f32),
            pltpu.VMEM((CS, D), f32),
            pltpu.VMEM_SHARED((N, D), f32),
            pltpu.SemaphoreType.DMA,
        ],
    )
    def k(vals_hbm, row_hbm, zeros_hbm, out_hbm, i0, i1, v0, v1, acc, sem):
        cid = lax.axis_index("c")
        sid = lax.axis_index("s")
        wid = sid * NC + cid
        pltpu.sync_copy(zeros_hbm.at[pl.ds(sid * NPT, NPT)],
                        acc.at[pl.ds(sid * NPT, NPT)])

        @pl.when(sid == NS - 1)
        def _():
            pltpu.sync_copy(zeros_hbm.at[pl.ds(NS * NPT, NREM)],
                            acc.at[pl.ds(NS * NPT, NREM)])

        plsc.subcore_barrier()
        base = wid * ESW
        gbase = sl * ES + base
        pltpu.sync_copy(row_hbm.at[pl.ds(gbase, CS)], i0)
        pltpu.sync_copy(vals_hbm.at[pl.ds(base, CS)], v0)

        def body(i, carry):
            c1 = (2 * i + 1) * CS
            c2 = c1 + CS
            l1a = pltpu.async_copy(row_hbm.at[pl.ds(gbase + c1, CS)], i1, sem)
            l1b = pltpu.async_copy(vals_hbm.at[pl.ds(base + c1, CS)], v1, sem)
            pltpu.sync_copy(v0, acc.at[i0], add=True)
            l1a.wait()
            l1b.wait()
            l0a = pltpu.async_copy(row_hbm.at[pl.ds(gbase + c2, CS)], i0, sem)
            l0b = pltpu.async_copy(vals_hbm.at[pl.ds(base + c2, CS)], v0, sem)
            pltpu.sync_copy(v1, acc.at[i1], add=True)
            l0a.wait()
            l0b.wait()
            return carry

        lax.fori_loop(0, NPAIR, body, 0)
        pltpu.sync_copy(v0, acc.at[i0], add=True)
        plsc.subcore_barrier()
        pltpu.sync_copy(acc.at[pl.ds(sid * NPT, NPT)],
                        out_hbm.at[cid, pl.ds(sid * NPT, NPT)])

        @pl.when(sid == NS - 1)
        def _():
            pltpu.sync_copy(acc.at[pl.ds(NS * NPT, NREM)],
                            out_hbm.at[cid, pl.ds(NS * NPT, NREM)])

    return k(vals, row, zeros)


BE = 4000


def _edge_specs(sl):
    off = sl * (ES // BE)
    return [
        pl.BlockSpec((BE, H), lambda i: (i, 0)),
        pl.BlockSpec((BE, 1), lambda i: (i + off, 0)),
        pl.BlockSpec((BE, 1), lambda i: (i + off, 0)),
        pl.BlockSpec((H, H), lambda i: (0, 0)),
        pl.BlockSpec((8, H), lambda i: (0, 0)),
    ]


def _gcl_edge_body(g, rad, ea, w2t, aux, out):
    radial = rad[...]
    pre = (g[...].astype(f32)
           + radial * aux[0:1, :]
           + ea[...] * aux[1:2, :])
    m1 = pre * jax.nn.sigmoid(pre)
    z = jnp.dot(m1, w2t[...], preferred_element_type=f32) + aux[2:3, :]
    m2 = z * jax.nn.sigmoid(z)
    att = jax.nn.sigmoid(
        jnp.sum(m2 * aux[3:4, :], axis=1, keepdims=True) + aux[4, 0])
    out[...] = m2 * att


def _gcl_edge(g, rad, ea, w2t, aux, sl):
    return pl.pallas_call(
        _gcl_edge_body,
        grid=(ES // BE,),
        in_specs=_edge_specs(sl),
        out_specs=pl.BlockSpec((BE, H), lambda i: (i, 0)),
        out_shape=jax.ShapeDtypeStruct((ES, H), f32),
    )(g, rad, ea, w2t, aux)


def _eq_edge_body(g, rad, ea, w2t, aux, out):
    radial = rad[...]
    pre = (g[...].astype(f32)
           + radial * aux[0:1, :]
           + ea[...] * aux[1:2, :])
    t1 = pre * jax.nn.sigmoid(pre)
    z = jnp.dot(t1, w2t[...], preferred_element_type=f32) + aux[2:3, :]
    t2 = z * jax.nn.sigmoid(z)
    phi = jnp.sum(t2 * aux[3:4, :], axis=1, keepdims=True)
    out[...] = phi / (jnp.sqrt(radial + 1e-8) + NORM_CONST)


def _eq_edge(g, rad, ea, w2t, aux, sl):
    return pl.pallas_call(
        _eq_edge_body,
        grid=(ES // BE,),
        in_specs=_edge_specs(sl),
        out_specs=pl.BlockSpec((BE, 1), lambda i: (i, 0)),
        out_shape=jax.ShapeDtypeStruct((ES, 1), f32),
    )(g, rad, ea, w2t, aux)


BN = 2000


def _proj_body(h, wrt, wct, aux, a_out, b_out):
    hv = h[...]
    a_out[...] = jnp.dot(hv, wrt[...], preferred_element_type=f32) + aux[0:1, :]
    b_out[...] = jnp.dot(hv, wct[...], preferred_element_type=f32)


def _proj2(h, wrt, wct, aux):
    return pl.pallas_call(
        _proj_body,
        grid=(N // BN,),
        in_specs=[
            pl.BlockSpec((BN, H), lambda i: (i, 0)),
            pl.BlockSpec((H, H), lambda i: (0, 0)),
            pl.BlockSpec((H, H), lambda i: (0, 0)),
            pl.BlockSpec((8, H), lambda i: (0, 0)),
        ],
        out_specs=[pl.BlockSpec((BN, H), lambda i: (i, 0)),
                   pl.BlockSpec((BN, H), lambda i: (i, 0))],
        out_shape=[jax.ShapeDtypeStruct((N, H), f32),
                   jax.ShapeDtypeStruct((N, H), f32)],
    )(h, wrt, wct, aux)


def _node_body(h, p0, p1, w1ht, w1at, w2t, wrnt, wcnt, aux,
               hn_out, an_out, bn_out):
    hv = h[...]
    agg = (p0[0] + p0[1] + p1[0] + p1[1]) * (1.0 / NORM_FACTOR)
    z = (jnp.dot(hv, w1ht[...], preferred_element_type=f32)
         + jnp.dot(agg, w1at[...], preferred_element_type=f32)
         + aux[0:1, :])
    t = z * jax.nn.sigmoid(z)
    hn = hv + jnp.dot(t, w2t[...], preferred_element_type=f32) + aux[1:2, :]
    hn_out[...] = hn
    an_out[...] = jnp.dot(hn, wrnt[...], preferred_element_type=f32) + aux[2:3, :]
    bn_out[...] = jnp.dot(hn, wcnt[...], preferred_element_type=f32)


def _node(h, p0, p1, w1ht, w1at, w2t, wrnt, wcnt, aux):
    wspec = pl.BlockSpec((H, H), lambda i: (0, 0))
    pspec = pl.BlockSpec((NC, BN, H), lambda i: (0, i, 0))
    return pl.pallas_call(
        _node_body,
        grid=(N // BN,),
        in_specs=[
            pl.BlockSpec((BN, H), lambda i: (i, 0)),
            pspec, pspec,
            wspec, wspec, wspec, wspec, wspec,
            pl.BlockSpec((8, H), lambda i: (0, 0)),
        ],
        out_specs=[pl.BlockSpec((BN, H), lambda i: (i, 0))] * 3,
        out_shape=[jax.ShapeDtypeStruct((N, H), f32)] * 3,
    )(h, p0, p1, w1ht, w1at, w2t, wrnt, wcnt, aux)


def _phase_weights(W1, b1):
    wrt = W1[:, :H].T
    wct = W1[:, H:2 * H].T
    wr = W1[:, 2 * H]
    wa = W1[:, 2 * H + 1]
    return wrt, wct, wr, wa, b1


def _pack8(*rows):
    aux = jnp.zeros((8, H), f32)
    for i, r in enumerate(rows):
        aux = aux.at[i, :r.shape[0]].set(r)
    return aux


def kernel(h, x, edge_attr, params, edge_index):
    row = edge_index[0].astype(i32)
    col = edge_index[1].astype(i32)
    ea = edge_attr.astype(f32)

    xs0, xs1, xs2 = x[:, 0], x[:, 1], x[:, 2]
    radial = _radial_kernel(xs0, xs1, xs2, row, col).reshape(E, 1)

    zeros128 = jnp.zeros((N, H), f32)
    zeros_n = jnp.zeros((N,), f32)

    g0, g1 = params["gcl"][0], params["gcl"][1]
    q = params["eq"]
    wrt0, wct0, wr0, wa0, b10 = _phase_weights(g0["e_w1"], g0["e_b1"])
    wrt1, wct1, wr1, wa1, b11 = _phase_weights(g1["e_w1"], g1["e_b1"])
    wrtq, wctq, wrq, waq, b1q = _phase_weights(q["w1"], q["b1"])

    A, B = _proj2(h, wrt0, wct0, _pack8(b10))

    for l, (p, wr, wa, nxt) in enumerate([
        (g0, wr0, wa0, (wrt1, wct1, b11)),
        (g1, wr1, wa1, (wrtq, wctq, b1q)),
    ]):
        aux_e = _pack8(wr, wa, p["e_b2"], p["a_w"][0], p["a_b"])
        w2t = p["e_w2"].T
        Ps = []
        for sl in range(NSLICE):
            G = _gather_add(A, B, row, col, sl)
            ef = _gcl_edge(G, radial, ea, w2t, aux_e, sl)
            Ps.append(_scatter_add(ef, row, zeros128, sl))
        h, A, B = _node(
            h, Ps[0], Ps[1],
            p["n_w1"][:, :H].T, p["n_w1"][:, H:].T, p["n_w2"].T,
            nxt[0], nxt[1], _pack8(p["n_b1"], p["n_b2"], nxt[2]))

    aux_q = _pack8(wrq, waq, q["b2"], q["w3"][0])
    w2tq = q["w2"].T
    ss = []
    for sl in range(NSLICE):
        G = _gather_add(A, B, row, col, sl)
        ss.append(_eq_edge(G, radial, ea, w2tq, aux_q, sl))
    s = jnp.concatenate(ss, axis=0)
    p0, p1, p2 = _eq_scatter(s.reshape(E), row, col, xs0, xs1, xs2, zeros_n)
    aggx = jnp.stack(
        [p0[:N] + p0[N:], p1[:N] + p1[N:], p2[:N] + p2[N:]], axis=1)
    x_out = x + aggx * (1.0 / NORM_FACTOR)
    return h, x_out


# speedup vs baseline: 1.1078x; 1.0035x over previous
import functools

import jax
import jax.numpy as jnp
from jax import lax
from jax.experimental import pallas as pl
from jax.experimental.pallas import tpu as pltpu
from jax.experimental.pallas import tpu_sc as plsc

N = 10000
E = 320000
H = 128
NORM_FACTOR = 100.0
NORM_CONST = 1.0

NC = 2
NS = 16
NW = NC * NS
EPW = E // NW
NSLICE = 2
ES = E // NSLICE
ESW = ES // NW
CHUNK = 400
NPT = 624
NREM = N - NS * NPT

f32 = jnp.float32
bf16 = jnp.bfloat16
i32 = jnp.int32


def _mesh():
    return plsc.VectorSubcoreMesh(core_axis_name="c", subcore_axis_name="s")


def _gather_add(A, B, row, col, sl):
    C = 200
    NCHUNK = ESW // C
    NPAIR = (NCHUNK - 1) // 2

    @functools.partial(
        pl.kernel,
        mesh=_mesh(),
        compiler_params=pltpu.CompilerParams(needs_layout_passes=False),
        out_type=jax.ShapeDtypeStruct((ES, H), f32),
        scratch_types=[
            pltpu.VMEM((ESW,), i32), pltpu.VMEM((ESW,), i32),
            pltpu.VMEM((C, H), f32), pltpu.VMEM((C, H), f32),
            pltpu.VMEM((C, H), f32), pltpu.VMEM((C, H), f32),
            pltpu.SemaphoreType.DMA, pltpu.SemaphoreType.DMA,
        ],
    )
    def k(a_hbm, b_hbm, row_hbm, col_hbm, g_hbm,
          rv, cv, a0, b0, a1, b1, gsem, wsem):
        wid = lax.axis_index("s") * NC + lax.axis_index("c")
        base = wid * ESW
        gbase = sl * ES + base
        pltpu.sync_copy(row_hbm.at[pl.ds(gbase, ESW)], rv)
        pltpu.sync_copy(col_hbm.at[pl.ds(gbase, ESW)], cv)

        def addloop(abuf, bbuf):
            def arow(r, c2):
                for j in range(H // 16):
                    slc = pl.ds(j * 16, 16)
                    abuf[r, slc] = abuf[r, slc] + bbuf[r, slc]
                return c2

            lax.fori_loop(0, C, arow, 0)

        def body(i, carry):
            loc0 = (2 * i) * C
            loc1 = loc0 + C
            off0 = base + loc0
            off1 = base + loc1

            @pl.when(i > 0)
            def _():
                pltpu.make_async_copy(a0, g_hbm.at[pl.ds(off0, C)], wsem).wait()
                pltpu.make_async_copy(a1, g_hbm.at[pl.ds(off1, C)], wsem).wait()

            g0a = pltpu.async_copy(a_hbm.at[rv.at[pl.ds(loc0, C)]], a0, gsem)
            g0b = pltpu.async_copy(b_hbm.at[cv.at[pl.ds(loc0, C)]], b0, gsem)
            g0a.wait()
            g0b.wait()
            g1a = pltpu.async_copy(a_hbm.at[rv.at[pl.ds(loc1, C)]], a1, gsem)
            g1b = pltpu.async_copy(b_hbm.at[cv.at[pl.ds(loc1, C)]], b1, gsem)
            addloop(a0, b0)
            pltpu.async_copy(a0, g_hbm.at[pl.ds(off0, C)], wsem)
            g1a.wait()
            g1b.wait()
            addloop(a1, b1)
            pltpu.async_copy(a1, g_hbm.at[pl.ds(off1, C)], wsem)
            return carry

        lax.fori_loop(0, NPAIR, body, 0)
        tloc = (2 * NPAIR) * C
        pltpu.make_async_copy(a0, g_hbm.at[pl.ds(base, C)], wsem).wait()
        ta = pltpu.async_copy(a_hbm.at[rv.at[pl.ds(tloc, C)]], a0, gsem)
        tb = pltpu.async_copy(b_hbm.at[cv.at[pl.ds(tloc, C)]], b0, gsem)
        ta.wait()
        tb.wait()
        addloop(a0, b0)
        pltpu.sync_copy(a0, g_hbm.at[pl.ds(base + tloc, C)])
        pltpu.make_async_copy(a1, g_hbm.at[pl.ds(base, C)], wsem).wait()

    return k(A, B, row, col)


def _radial_kernel(xs0, xs1, xs2, row, col):
    @functools.partial(
        pl.kernel,
        mesh=_mesh(),
        compiler_params=pltpu.CompilerParams(needs_layout_passes=False),
        out_type=jax.ShapeDtypeStruct((E,), f32),
        scratch_types=[
            pltpu.VMEM((N,), f32),
            pltpu.VMEM((N,), f32),
            pltpu.VMEM((N,), f32),
            pltpu.VMEM((CHUNK,), i32),
            pltpu.VMEM((CHUNK,), i32),
            pltpu.VMEM((CHUNK,), f32),
        ],
    )
    def k(x0_hbm, x1_hbm, x2_hbm, row_hbm, col_hbm, r_hbm,
          t0, t1, t2, rv, cv, rbuf):
        wid = lax.axis_index("s") * NC + lax.axis_index("c")
        pltpu.sync_copy(x0_hbm, t0)
        pltpu.sync_copy(x1_hbm, t1)
        pltpu.sync_copy(x2_hbm, t2)
        base = wid * EPW

        def body(i, carry):
            off = base + i * CHUNK
            pltpu.sync_copy(row_hbm.at[pl.ds(off, CHUNK)], rv)
            pltpu.sync_copy(col_hbm.at[pl.ds(off, CHUNK)], cv)

            def grp(j, c2):
                ridx = rv[pl.ds(j * 16, 16)]
                cidx = cv[pl.ds(j * 16, 16)]
                d0 = plsc.load_gather(t0, [ridx]) - plsc.load_gather(t0, [cidx])
                d1 = plsc.load_gather(t1, [ridx]) - plsc.load_gather(t1, [cidx])
                d2 = plsc.load_gather(t2, [ridx]) - plsc.load_gather(t2, [cidx])
                rbuf[pl.ds(j * 16, 16)] = d0 * d0 + d1 * d1 + d2 * d2
                return c2

            lax.fori_loop(0, CHUNK // 16, grp, 0)
            pltpu.sync_copy(rbuf, r_hbm.at[pl.ds(off, CHUNK)])
            return carry

        lax.fori_loop(0, EPW // CHUNK, body, 0)

    return k(xs0, xs1, xs2, row, col)


def _eq_scatter(s, row, col, xs0, xs1, xs2, zeros_n):
    @functools.partial(
        pl.kernel,
        mesh=_mesh(),
        compiler_params=pltpu.CompilerParams(needs_layout_passes=False),
        out_type=[jax.ShapeDtypeStruct((NC * N,), f32)] * 3,
        scratch_types=[
            pltpu.VMEM((N,), f32),
            pltpu.VMEM((N,), f32),
            pltpu.VMEM((N,), f32),
            pltpu.VMEM((CHUNK,), i32),
            pltpu.VMEM((CHUNK,), i32),
            pltpu.VMEM((CHUNK,), f32),
            pltpu.VMEM((CHUNK,), f32),
            pltpu.VMEM((CHUNK,), f32),
            pltpu.VMEM((CHUNK,), f32),
            pltpu.VMEM((N,), f32),
            pltpu.VMEM_SHARED((N,), f32),
            pltpu.VMEM_SHARED((N,), f32),
            pltpu.VMEM_SHARED((N,), f32),
        ],
    )
    def k(s_hbm, row_hbm, col_hbm, x0_hbm, x1_hbm, x2_hbm, z_hbm,
          o0_hbm, o1_hbm, o2_hbm,
          t0, t1, t2, rv, cv, sv, f0, f1, f2, stage, acc0, acc1, acc2):
        cid = lax.axis_index("c")
        sid = lax.axis_index("s")
        wid = sid * NC + cid
        pltpu.sync_copy(x0_hbm, t0)
        pltpu.sync_copy(x1_hbm, t1)
        pltpu.sync_copy(x2_hbm, t2)

        accs = [acc0, acc1, acc2]
        for kk in range(3):
            @pl.when(sid == kk)
            def _(kk=kk):
                pltpu.sync_copy(z_hbm, stage)
                pltpu.sync_copy(stage, accs[kk])

        plsc.subcore_barrier()
        base = wid * EPW

        def body(i, carry):
            off = base + i * CHUNK
            pltpu.sync_copy(row_hbm.at[pl.ds(off, CHUNK)], rv)
            pltpu.sync_copy(col_hbm.at[pl.ds(off, CHUNK)], cv)
            pltpu.sync_copy(s_hbm.at[pl.ds(off, CHUNK)], sv)

            def grp(j, c2):
                sl = pl.ds(j * 16, 16)
                ridx = rv[sl]
                cidx = cv[sl]
                se = sv[sl]
                f0[sl] = (plsc.load_gather(t0, [ridx])
                          - plsc.load_gather(t0, [cidx])) * se
                f1[sl] = (plsc.load_gather(t1, [ridx])
                          - plsc.load_gather(t1, [cidx])) * se
                f2[sl] = (plsc.load_gather(t2, [ridx])
                          - plsc.load_gather(t2, [cidx])) * se
                return c2

            lax.fori_loop(0, CHUNK // 16, grp, 0)
            pltpu.sync_copy(f0, acc0.at[rv], add=True)
            pltpu.sync_copy(f1, acc1.at[rv], add=True)
            pltpu.sync_copy(f2, acc2.at[rv], add=True)
            return carry

        lax.fori_loop(0, EPW // CHUNK, body, 0)
        plsc.subcore_barrier()

        outs = [o0_hbm, o1_hbm, o2_hbm]
        for kk in range(3):
            @pl.when(sid == kk)
            def _(kk=kk):
                pltpu.sync_copy(accs[kk], stage)
                pltpu.sync_copy(stage, outs[kk].at[pl.ds(cid * N, N)])

    return k(s, row, col, xs0, xs1, xs2, zeros_n)


def _scatter_add(vals, row, zeros, sl, D=H, CS=40):
    NPAIR = (ESW // CS - 1) // 2

    @functools.partial(
        pl.kernel,
        mesh=_mesh(),
        out_type=jax.ShapeDtypeStruct((NC, N, D), f32),
        scratch_types=[
            pltpu.VMEM((CS,), i32),
            pltpu.VMEM((CS,), i32),
            pltpu.VMEM((CS, D), f32),
            pltpu.VMEM((CS, D), f32),
            pltpu.VMEM_SHARED((N, D), f32),
            pltpu.SemaphoreType.DMA,
        ],
    )
    def k(vals_hbm, row_hbm, zeros_hbm, out_hbm, i0, i1, v0, v1, acc, sem):
        cid = lax.axis_index("c")
        sid = lax.axis_index("s")
        wid = sid * NC + cid
        pltpu.sync_copy(zeros_hbm.at[pl.ds(sid * NPT, NPT)],
                        acc.at[pl.ds(sid * NPT, NPT)])

        @pl.when(sid == NS - 1)
        def _():
            pltpu.sync_copy(zeros_hbm.at[pl.ds(NS * NPT, NREM)],
                            acc.at[pl.ds(NS * NPT, NREM)])

        plsc.subcore_barrier()
        base = wid * ESW
        gbase = sl * ES + base
        pltpu.sync_copy(row_hbm.at[pl.ds(gbase, CS)], i0)
        pltpu.sync_copy(vals_hbm.at[pl.ds(base, CS)], v0)

        def body(i, carry):
            c1 = (2 * i + 1) * CS
            c2 = c1 + CS
            l1a = pltpu.async_copy(row_hbm.at[pl.ds(gbase + c1, CS)], i1, sem)
            l1b = pltpu.async_copy(vals_hbm.at[pl.ds(base + c1, CS)], v1, sem)
            pltpu.sync_copy(v0, acc.at[i0], add=True)
            l1a.wait()
            l1b.wait()
            l0a = pltpu.async_copy(row_hbm.at[pl.ds(gbase + c2, CS)], i0, sem)
            l0b = pltpu.async_copy(vals_hbm.at[pl.ds(base + c2, CS)], v0, sem)
            pltpu.sync_copy(v1, acc.at[i1], add=True)
            l0a.wait()
            l0b.wait()
            return carry

        lax.fori_loop(0, NPAIR, body, 0)
        pltpu.sync_copy(v0, acc.at[i0], add=True)
        plsc.subcore_barrier()
        pltpu.sync_copy(acc.at[pl.ds(sid * NPT, NPT)],
                        out_hbm.at[cid, pl.ds(sid * NPT, NPT)])

        @pl.when(sid == NS - 1)
        def _():
            pltpu.sync_copy(acc.at[pl.ds(NS * NPT, NREM)],
                            out_hbm.at[cid, pl.ds(NS * NPT, NREM)])

    return k(vals, row, zeros)


BE = 8000


def _edge_specs(sl):
    off = sl * (ES // BE)
    return [
        pl.BlockSpec((BE, H), lambda i: (i, 0)),
        pl.BlockSpec((BE, 1), lambda i: (i + off, 0)),
        pl.BlockSpec((BE, 1), lambda i: (i + off, 0)),
        pl.BlockSpec((H, H), lambda i: (0, 0)),
        pl.BlockSpec((8, H), lambda i: (0, 0)),
    ]


def _gcl_edge_body(g, rad, ea, w2t, aux, out):
    radial = rad[...]
    pre = (g[...].astype(f32)
           + radial * aux[0:1, :]
           + ea[...] * aux[1:2, :])
    m1 = pre * jax.nn.sigmoid(pre)
    z = jnp.dot(m1, w2t[...], preferred_element_type=f32) + aux[2:3, :]
    m2 = z * jax.nn.sigmoid(z)
    att = jax.nn.sigmoid(
        jnp.sum(m2 * aux[3:4, :], axis=1, keepdims=True) + aux[4, 0])
    out[...] = m2 * att


def _gcl_edge(g, rad, ea, w2t, aux, sl):
    return pl.pallas_call(
        _gcl_edge_body,
        grid=(ES // BE,),
        in_specs=_edge_specs(sl),
        out_specs=pl.BlockSpec((BE, H), lambda i: (i, 0)),
        out_shape=jax.ShapeDtypeStruct((ES, H), f32),
    )(g, rad, ea, w2t, aux)


def _eq_edge_body(g, rad, ea, w2t, aux, out):
    radial = rad[...]
    pre = (g[...].astype(f32)
           + radial * aux[0:1, :]
           + ea[...] * aux[1:2, :])
    t1 = pre * jax.nn.sigmoid(pre)
    z = jnp.dot(t1, w2t[...], preferred_element_type=f32) + aux[2:3, :]
    t2 = z * jax.nn.sigmoid(z)
    phi = jnp.sum(t2 * aux[3:4, :], axis=1, keepdims=True)
    out[...] = phi / (jnp.sqrt(radial + 1e-8) + NORM_CONST)


def _eq_edge(g, rad, ea, w2t, aux, sl):
    return pl.pallas_call(
        _eq_edge_body,
        grid=(ES // BE,),
        in_specs=_edge_specs(sl),
        out_specs=pl.BlockSpec((BE, 1), lambda i: (i, 0)),
        out_shape=jax.ShapeDtypeStruct((ES, 1), f32),
    )(g, rad, ea, w2t, aux)


BN = 2000


def _proj_body(h, wrt, wct, aux, a_out, b_out):
    hv = h[...]
    a_out[...] = jnp.dot(hv, wrt[...], preferred_element_type=f32) + aux[0:1, :]
    b_out[...] = jnp.dot(hv, wct[...], preferred_element_type=f32)


def _proj2(h, wrt, wct, aux):
    return pl.pallas_call(
        _proj_body,
        grid=(N // BN,),
        in_specs=[
            pl.BlockSpec((BN, H), lambda i: (i, 0)),
            pl.BlockSpec((H, H), lambda i: (0, 0)),
            pl.BlockSpec((H, H), lambda i: (0, 0)),
            pl.BlockSpec((8, H), lambda i: (0, 0)),
        ],
        out_specs=[pl.BlockSpec((BN, H), lambda i: (i, 0)),
                   pl.BlockSpec((BN, H), lambda i: (i, 0))],
        out_shape=[jax.ShapeDtypeStruct((N, H), f32),
                   jax.ShapeDtypeStruct((N, H), f32)],
    )(h, wrt, wct, aux)


def _node_body(h, p0, p1, w1ht, w1at, w2t, wrnt, wcnt, aux,
               hn_out, an_out, bn_out):
    hv = h[...]
    agg = (p0[0] + p0[1] + p1[0] + p1[1]) * (1.0 / NORM_FACTOR)
    z = (jnp.dot(hv, w1ht[...], preferred_element_type=f32)
         + jnp.dot(agg, w1at[...], preferred_element_type=f32)
         + aux[0:1, :])
    t = z * jax.nn.sigmoid(z)
    hn = hv + jnp.dot(t, w2t[...], preferred_element_type=f32) + aux[1:2, :]
    hn_out[...] = hn
    an_out[...] = jnp.dot(hn, wrnt[...], preferred_element_type=f32) + aux[2:3, :]
    bn_out[...] = jnp.dot(hn, wcnt[...], preferred_element_type=f32)


def _node(h, p0, p1, w1ht, w1at, w2t, wrnt, wcnt, aux):
    wspec = pl.BlockSpec((H, H), lambda i: (0, 0))
    pspec = pl.BlockSpec((NC, BN, H), lambda i: (0, i, 0))
    return pl.pallas_call(
        _node_body,
        grid=(N // BN,),
        in_specs=[
            pl.BlockSpec((BN, H), lambda i: (i, 0)),
            pspec, pspec,
            wspec, wspec, wspec, wspec, wspec,
            pl.BlockSpec((8, H), lambda i: (0, 0)),
        ],
        out_specs=[pl.BlockSpec((BN, H), lambda i: (i, 0))] * 3,
        out_shape=[jax.ShapeDtypeStruct((N, H), f32)] * 3,
    )(h, p0, p1, w1ht, w1at, w2t, wrnt, wcnt, aux)


def _phase_weights(W1, b1):
    wrt = W1[:, :H].T
    wct = W1[:, H:2 * H].T
    wr = W1[:, 2 * H]
    wa = W1[:, 2 * H + 1]
    return wrt, wct, wr, wa, b1


def _pack8(*rows):
    aux = jnp.zeros((8, H), f32)
    for i, r in enumerate(rows):
        aux = aux.at[i, :r.shape[0]].set(r)
    return aux


def kernel(h, x, edge_attr, params, edge_index):
    row = edge_index[0].astype(i32)
    col = edge_index[1].astype(i32)
    ea = edge_attr.astype(f32)

    xs0, xs1, xs2 = x[:, 0], x[:, 1], x[:, 2]
    radial = _radial_kernel(xs0, xs1, xs2, row, col).reshape(E, 1)

    zeros128 = jnp.zeros((N, H), f32)
    zeros_n = jnp.zeros((N,), f32)

    g0, g1 = params["gcl"][0], params["gcl"][1]
    q = params["eq"]
    wrt0, wct0, wr0, wa0, b10 = _phase_weights(g0["e_w1"], g0["e_b1"])
    wrt1, wct1, wr1, wa1, b11 = _phase_weights(g1["e_w1"], g1["e_b1"])
    wrtq, wctq, wrq, waq, b1q = _phase_weights(q["w1"], q["b1"])

    A, B = _proj2(h, wrt0, wct0, _pack8(b10))

    for l, (p, wr, wa, nxt) in enumerate([
        (g0, wr0, wa0, (wrt1, wct1, b11)),
        (g1, wr1, wa1, (wrtq, wctq, b1q)),
    ]):
        aux_e = _pack8(wr, wa, p["e_b2"], p["a_w"][0], p["a_b"])
        w2t = p["e_w2"].T
        Ps = []
        for sl in range(NSLICE):
            G = _gather_add(A, B, row, col, sl)
            ef = _gcl_edge(G, radial, ea, w2t, aux_e, sl)
            Ps.append(_scatter_add(ef, row, zeros128, sl))
        h, A, B = _node(
            h, Ps[0], Ps[1],
            p["n_w1"][:, :H].T, p["n_w1"][:, H:].T, p["n_w2"].T,
            nxt[0], nxt[1], _pack8(p["n_b1"], p["n_b2"], nxt[2]))

    aux_q = _pack8(wrq, waq, q["b2"], q["w3"][0])
    w2tq = q["w2"].T
    ss = []
    for sl in range(NSLICE):
        G = _gather_add(A, B, row, col, sl)
        ss.append(_eq_edge(G, radial, ea, w2tq, aux_q, sl))
    s = jnp.concatenate(ss, axis=0)
    p0, p1, p2 = _eq_scatter(s.reshape(E), row, col, xs0, xs1, xs2, zeros_n)
    aggx = jnp.stack(
        [p0[:N] + p0[N:], p1[:N] + p1[N:], p2[:N] + p2[N:]], axis=1)
    x_out = x + aggx * (1.0 / NORM_FACTOR)
    return h, x_out


# speedup vs baseline: 1.1436x; 1.0323x over previous
import functools

import jax
import jax.numpy as jnp
from jax import lax
from jax.experimental import pallas as pl
from jax.experimental.pallas import tpu as pltpu
from jax.experimental.pallas import tpu_sc as plsc

N = 10000
E = 320000
H = 128
NORM_FACTOR = 100.0
NORM_CONST = 1.0

NC = 2
NS = 16
NW = NC * NS
EPW = E // NW
NSLICE = 2
ES = E // NSLICE
ESW = ES // NW
CHUNK = 400
NPT = 624
NREM = N - NS * NPT

f32 = jnp.float32
bf16 = jnp.bfloat16
i32 = jnp.int32


def _mesh():
    return plsc.VectorSubcoreMesh(core_axis_name="c", subcore_axis_name="s")


def _gather_add(A, B, row, col, sl):
    C = 200
    NCHUNK = ESW // C
    NPAIR = (NCHUNK - 1) // 2

    @functools.partial(
        pl.kernel,
        mesh=_mesh(),
        compiler_params=pltpu.CompilerParams(needs_layout_passes=False),
        out_type=jax.ShapeDtypeStruct((ES, H), f32),
        scratch_types=[
            pltpu.VMEM((ESW,), i32), pltpu.VMEM((ESW,), i32),
            pltpu.VMEM((C, H), f32), pltpu.VMEM((C, H), f32),
            pltpu.VMEM((C, H), f32), pltpu.VMEM((C, H), f32),
            pltpu.SemaphoreType.DMA, pltpu.SemaphoreType.DMA,
        ],
    )
    def k(a_hbm, b_hbm, row_hbm, col_hbm, g_hbm,
          rv, cv, a0, b0, a1, b1, gsem, wsem):
        wid = lax.axis_index("s") * NC + lax.axis_index("c")
        base = wid * ESW
        gbase = sl * ES + base
        pltpu.sync_copy(row_hbm.at[pl.ds(gbase, ESW)], rv)
        pltpu.sync_copy(col_hbm.at[pl.ds(gbase, ESW)], cv)

        def addloop(abuf, bbuf):
            def arow(r, c2):
                for j in range(H // 16):
                    slc = pl.ds(j * 16, 16)
                    abuf[r, slc] = abuf[r, slc] + bbuf[r, slc]
                return c2

            lax.fori_loop(0, C, arow, 0)

        def body(i, carry):
            loc0 = (2 * i) * C
            loc1 = loc0 + C
            off0 = base + loc0
            off1 = base + loc1

            @pl.when(i > 0)
            def _():
                pltpu.make_async_copy(a0, g_hbm.at[pl.ds(off0, C)], wsem).wait()
                pltpu.make_async_copy(a1, g_hbm.at[pl.ds(off1, C)], wsem).wait()

            g0a = pltpu.async_copy(a_hbm.at[rv.at[pl.ds(loc0, C)]], a0, gsem)
            g0b = pltpu.async_copy(b_hbm.at[cv.at[pl.ds(loc0, C)]], b0, gsem)
            g0a.wait()
            g0b.wait()
            g1a = pltpu.async_copy(a_hbm.at[rv.at[pl.ds(loc1, C)]], a1, gsem)
            g1b = pltpu.async_copy(b_hbm.at[cv.at[pl.ds(loc1, C)]], b1, gsem)
            addloop(a0, b0)
            pltpu.async_copy(a0, g_hbm.at[pl.ds(off0, C)], wsem)
            g1a.wait()
            g1b.wait()
            addloop(a1, b1)
            pltpu.async_copy(a1, g_hbm.at[pl.ds(off1, C)], wsem)
            return carry

        lax.fori_loop(0, NPAIR, body, 0)
        tloc = (2 * NPAIR) * C
        pltpu.make_async_copy(a0, g_hbm.at[pl.ds(base, C)], wsem).wait()
        ta = pltpu.async_copy(a_hbm.at[rv.at[pl.ds(tloc, C)]], a0, gsem)
        tb = pltpu.async_copy(b_hbm.at[cv.at[pl.ds(tloc, C)]], b0, gsem)
        ta.wait()
        tb.wait()
        addloop(a0, b0)
        pltpu.sync_copy(a0, g_hbm.at[pl.ds(base + tloc, C)])
        pltpu.make_async_copy(a1, g_hbm.at[pl.ds(base, C)], wsem).wait()

    return k(A, B, row, col)


def _radial_kernel(xs0, xs1, xs2, row, col):
    @functools.partial(
        pl.kernel,
        mesh=_mesh(),
        compiler_params=pltpu.CompilerParams(needs_layout_passes=False),
        out_type=jax.ShapeDtypeStruct((E,), f32),
        scratch_types=[
            pltpu.VMEM((N,), f32),
            pltpu.VMEM((N,), f32),
            pltpu.VMEM((N,), f32),
            pltpu.VMEM((CHUNK,), i32),
            pltpu.VMEM((CHUNK,), i32),
            pltpu.VMEM((CHUNK,), f32),
        ],
    )
    def k(x0_hbm, x1_hbm, x2_hbm, row_hbm, col_hbm, r_hbm,
          t0, t1, t2, rv, cv, rbuf):
        wid = lax.axis_index("s") * NC + lax.axis_index("c")
        pltpu.sync_copy(x0_hbm, t0)
        pltpu.sync_copy(x1_hbm, t1)
        pltpu.sync_copy(x2_hbm, t2)
        base = wid * EPW

        def body(i, carry):
            off = base + i * CHUNK
            pltpu.sync_copy(row_hbm.at[pl.ds(off, CHUNK)], rv)
            pltpu.sync_copy(col_hbm.at[pl.ds(off, CHUNK)], cv)

            def grp(j, c2):
                ridx = rv[pl.ds(j * 16, 16)]
                cidx = cv[pl.ds(j * 16, 16)]
                d0 = plsc.load_gather(t0, [ridx]) - plsc.load_gather(t0, [cidx])
                d1 = plsc.load_gather(t1, [ridx]) - plsc.load_gather(t1, [cidx])
                d2 = plsc.load_gather(t2, [ridx]) - plsc.load_gather(t2, [cidx])
                rbuf[pl.ds(j * 16, 16)] = d0 * d0 + d1 * d1 + d2 * d2
                return c2

            lax.fori_loop(0, CHUNK // 16, grp, 0)
            pltpu.sync_copy(rbuf, r_hbm.at[pl.ds(off, CHUNK)])
            return carry

        lax.fori_loop(0, EPW // CHUNK, body, 0)

    return k(xs0, xs1, xs2, row, col)


def _eq_scatter(s, row, col, xs0, xs1, xs2, zeros_n, sl):
    CH = 200

    @functools.partial(
        pl.kernel,
        mesh=_mesh(),
        compiler_params=pltpu.CompilerParams(needs_layout_passes=False),
        out_type=[jax.ShapeDtypeStruct((NC * N,), f32)] * 3,
        scratch_types=[
            pltpu.VMEM((N,), f32),
            pltpu.VMEM((N,), f32),
            pltpu.VMEM((N,), f32),
            pltpu.VMEM((CH,), i32),
            pltpu.VMEM((CH,), i32),
            pltpu.VMEM((CH,), f32),
            pltpu.VMEM((CH,), f32),
            pltpu.VMEM((CH,), f32),
            pltpu.VMEM((CH,), f32),
            pltpu.VMEM((N,), f32),
            pltpu.VMEM_SHARED((N,), f32),
            pltpu.VMEM_SHARED((N,), f32),
            pltpu.VMEM_SHARED((N,), f32),
        ],
    )
    def k(s_hbm, row_hbm, col_hbm, x0_hbm, x1_hbm, x2_hbm, z_hbm,
          o0_hbm, o1_hbm, o2_hbm,
          t0, t1, t2, rv, cv, sv, f0, f1, f2, stage, acc0, acc1, acc2):
        cid = lax.axis_index("c")
        sid = lax.axis_index("s")
        wid = sid * NC + cid
        pltpu.sync_copy(x0_hbm, t0)
        pltpu.sync_copy(x1_hbm, t1)
        pltpu.sync_copy(x2_hbm, t2)

        accs = [acc0, acc1, acc2]
        for kk in range(3):
            @pl.when(sid == kk)
            def _(kk=kk):
                pltpu.sync_copy(z_hbm, stage)
                pltpu.sync_copy(stage, accs[kk])

        plsc.subcore_barrier()
        base = wid * ESW
        gbase = sl * ES + base

        def body(i, carry):
            goff = gbase + i * CH
            soff = base + i * CH
            pltpu.sync_copy(row_hbm.at[pl.ds(goff, CH)], rv)
            pltpu.sync_copy(col_hbm.at[pl.ds(goff, CH)], cv)
            pltpu.sync_copy(s_hbm.at[pl.ds(soff, CH)], sv)

            def grp(j, c2):
                lanes = pl.ds(j * 16, 16)
                ridx = rv[lanes]
                cidx = cv[lanes]
                se = sv[lanes]
                f0[lanes] = (plsc.load_gather(t0, [ridx])
                             - plsc.load_gather(t0, [cidx])) * se
                f1[lanes] = (plsc.load_gather(t1, [ridx])
                             - plsc.load_gather(t1, [cidx])) * se
                f2[lanes] = (plsc.load_gather(t2, [ridx])
                             - plsc.load_gather(t2, [cidx])) * se
                return c2

            lax.fori_loop(0, CH // 16, grp, 0)
            pltpu.sync_copy(f0, acc0.at[rv], add=True)
            pltpu.sync_copy(f1, acc1.at[rv], add=True)
            pltpu.sync_copy(f2, acc2.at[rv], add=True)
            return carry

        lax.fori_loop(0, ESW // CH, body, 0)
        plsc.subcore_barrier()

        outs = [o0_hbm, o1_hbm, o2_hbm]
        for kk in range(3):
            @pl.when(sid == kk)
            def _(kk=kk):
                pltpu.sync_copy(accs[kk], stage)
                pltpu.sync_copy(stage, outs[kk].at[pl.ds(cid * N, N)])

    return k(s, row, col, xs0, xs1, xs2, zeros_n)


def _scatter_add(vals, row, zeros, sl, D=H, CS=40):
    NPAIR = (ESW // CS - 1) // 2

    @functools.partial(
        pl.kernel,
        mesh=_mesh(),
        out_type=jax.ShapeDtypeStruct((NC, N, D), f32),
        scratch_types=[
            pltpu.VMEM((CS,), i32),
            pltpu.VMEM((CS,), i32),
            pltpu.VMEM((CS, D), f32),
            pltpu.VMEM((CS, D), f32),
            pltpu.VMEM_SHARED((N, D), f32),
            pltpu.SemaphoreType.DMA,
        ],
    )
    def k(vals_hbm, row_hbm, zeros_hbm, out_hbm, i0, i1, v0, v1, acc, sem):
        cid = lax.axis_index("c")
        sid = lax.axis_index("s")
        wid = sid * NC + cid
        pltpu.sync_copy(zeros_hbm.at[pl.ds(sid * NPT, NPT)],
                        acc.at[pl.ds(sid * NPT, NPT)])

        @pl.when(sid == NS - 1)
        def _():
            pltpu.sync_copy(zeros_hbm.at[pl.ds(NS * NPT, NREM)],
                            acc.at[pl.ds(NS * NPT, NREM)])

        plsc.subcore_barrier()
        base = wid * ESW
        gbase = sl * ES + base
        pltpu.sync_copy(row_hbm.at[pl.ds(gbase, CS)], i0)
        pltpu.sync_copy(vals_hbm.at[pl.ds(base, CS)], v0)

        def body(i, carry):
            c1 = (2 * i + 1) * CS
            c2 = c1 + CS
            l1a = pltpu.async_copy(row_hbm.at[pl.ds(gbase + c1, CS)], i1, sem)
            l1b = pltpu.async_copy(vals_hbm.at[pl.ds(base + c1, CS)], v1, sem)
            pltpu.sync_copy(v0, acc.at[i0], add=True)
            l1a.wait()
            l1b.wait()
            l0a = pltpu.async_copy(row_hbm.at[pl.ds(gbase + c2, CS)], i0, sem)
            l0b = pltpu.async_copy(vals_hbm.at[pl.ds(base + c2, CS)], v0, sem)
            pltpu.sync_copy(v1, acc.at[i1], add=True)
            l0a.wait()
            l0b.wait()
            return carry

        lax.fori_loop(0, NPAIR, body, 0)
        pltpu.sync_copy(v0, acc.at[i0], add=True)
        plsc.subcore_barrier()
        pltpu.sync_copy(acc.at[pl.ds(sid * NPT, NPT)],
                        out_hbm.at[cid, pl.ds(sid * NPT, NPT)])

        @pl.when(sid == NS - 1)
        def _():
            pltpu.sync_copy(acc.at[pl.ds(NS * NPT, NREM)],
                            out_hbm.at[cid, pl.ds(NS * NPT, NREM)])

    return k(vals, row, zeros)


BE = 8000


def _edge_specs(sl):
    off = sl * (ES // BE)
    return [
        pl.BlockSpec((BE, H), lambda i: (i, 0)),
        pl.BlockSpec((BE, 1), lambda i: (i + off, 0)),
        pl.BlockSpec((BE, 1), lambda i: (i + off, 0)),
        pl.BlockSpec((H, H), lambda i: (0, 0)),
        pl.BlockSpec((8, H), lambda i: (0, 0)),
    ]


def _gcl_edge_body(g, rad, ea, w2t, aux, out):
    radial = rad[...]
    pre = (g[...].astype(f32)
           + radial * aux[0:1, :]
           + ea[...] * aux[1:2, :])
    m1 = pre * jax.nn.sigmoid(pre)
    z = jnp.dot(m1, w2t[...], preferred_element_type=f32) + aux[2:3, :]
    m2 = z * jax.nn.sigmoid(z)
    att = jax.nn.sigmoid(
        jnp.sum(m2 * aux[3:4, :], axis=1, keepdims=True) + aux[4, 0])
    out[...] = m2 * att


def _gcl_edge(g, rad, ea, w2t, aux, sl):
    return pl.pallas_call(
        _gcl_edge_body,
        grid=(ES // BE,),
        in_specs=_edge_specs(sl),
        out_specs=pl.BlockSpec((BE, H), lambda i: (i, 0)),
        out_shape=jax.ShapeDtypeStruct((ES, H), f32),
    )(g, rad, ea, w2t, aux)


def _eq_edge_body(g, rad, ea, w2t, aux, out):
    radial = rad[...]
    pre = (g[...].astype(f32)
           + radial * aux[0:1, :]
           + ea[...] * aux[1:2, :])
    t1 = pre * jax.nn.sigmoid(pre)
    z = jnp.dot(t1, w2t[...], preferred_element_type=f32) + aux[2:3, :]
    t2 = z * jax.nn.sigmoid(z)
    phi = jnp.sum(t2 * aux[3:4, :], axis=1, keepdims=True)
    out[...] = phi / (jnp.sqrt(radial + 1e-8) + NORM_CONST)


def _eq_edge(g, rad, ea, w2t, aux, sl):
    return pl.pallas_call(
        _eq_edge_body,
        grid=(ES // BE,),
        in_specs=_edge_specs(sl),
        out_specs=pl.BlockSpec((BE, 1), lambda i: (i, 0)),
        out_shape=jax.ShapeDtypeStruct((ES, 1), f32),
    )(g, rad, ea, w2t, aux)


BN = 2000


def _proj_body(h, wrt, wct, aux, a_out, b_out):
    hv = h[...]
    a_out[...] = jnp.dot(hv, wrt[...], preferred_element_type=f32) + aux[0:1, :]
    b_out[...] = jnp.dot(hv, wct[...], preferred_element_type=f32)


def _proj2(h, wrt, wct, aux):
    return pl.pallas_call(
        _proj_body,
        grid=(N // BN,),
        in_specs=[
            pl.BlockSpec((BN, H), lambda i: (i, 0)),
            pl.BlockSpec((H, H), lambda i: (0, 0)),
            pl.BlockSpec((H, H), lambda i: (0, 0)),
            pl.BlockSpec((8, H), lambda i: (0, 0)),
        ],
        out_specs=[pl.BlockSpec((BN, H), lambda i: (i, 0)),
                   pl.BlockSpec((BN, H), lambda i: (i, 0))],
        out_shape=[jax.ShapeDtypeStruct((N, H), f32),
                   jax.ShapeDtypeStruct((N, H), f32)],
    )(h, wrt, wct, aux)


def _node_body(h, p0, p1, w1ht, w1at, w2t, wrnt, wcnt, aux,
               hn_out, an_out, bn_out):
    hv = h[...]
    agg = (p0[0] + p0[1] + p1[0] + p1[1]) * (1.0 / NORM_FACTOR)
    z = (jnp.dot(hv, w1ht[...], preferred_element_type=f32)
         + jnp.dot(agg, w1at[...], preferred_element_type=f32)
         + aux[0:1, :])
    t = z * jax.nn.sigmoid(z)
    hn = hv + jnp.dot(t, w2t[...], preferred_element_type=f32) + aux[1:2, :]
    hn_out[...] = hn
    an_out[...] = jnp.dot(hn, wrnt[...], preferred_element_type=f32) + aux[2:3, :]
    bn_out[...] = jnp.dot(hn, wcnt[...], preferred_element_type=f32)


def _node(h, p0, p1, w1ht, w1at, w2t, wrnt, wcnt, aux):
    wspec = pl.BlockSpec((H, H), lambda i: (0, 0))
    pspec = pl.BlockSpec((NC, BN, H), lambda i: (0, i, 0))
    return pl.pallas_call(
        _node_body,
        grid=(N // BN,),
        in_specs=[
            pl.BlockSpec((BN, H), lambda i: (i, 0)),
            pspec, pspec,
            wspec, wspec, wspec, wspec, wspec,
            pl.BlockSpec((8, H), lambda i: (0, 0)),
        ],
        out_specs=[pl.BlockSpec((BN, H), lambda i: (i, 0))] * 3,
        out_shape=[jax.ShapeDtypeStruct((N, H), f32)] * 3,
    )(h, p0, p1, w1ht, w1at, w2t, wrnt, wcnt, aux)


def _phase_weights(W1, b1):
    wrt = W1[:, :H].T
    wct = W1[:, H:2 * H].T
    wr = W1[:, 2 * H]
    wa = W1[:, 2 * H + 1]
    return wrt, wct, wr, wa, b1


def _pack8(*rows):
    aux = jnp.zeros((8, H), f32)
    for i, r in enumerate(rows):
        aux = aux.at[i, :r.shape[0]].set(r)
    return aux


def kernel(h, x, edge_attr, params, edge_index):
    row = edge_index[0].astype(i32)
    col = edge_index[1].astype(i32)
    ea = edge_attr.astype(f32)

    xs0, xs1, xs2 = x[:, 0], x[:, 1], x[:, 2]
    radial = _radial_kernel(xs0, xs1, xs2, row, col).reshape(E, 1)

    zeros128 = jnp.zeros((N, H), f32)
    zeros_n = jnp.zeros((N,), f32)

    g0, g1 = params["gcl"][0], params["gcl"][1]
    q = params["eq"]
    wrt0, wct0, wr0, wa0, b10 = _phase_weights(g0["e_w1"], g0["e_b1"])
    wrt1, wct1, wr1, wa1, b11 = _phase_weights(g1["e_w1"], g1["e_b1"])
    wrtq, wctq, wrq, waq, b1q = _phase_weights(q["w1"], q["b1"])

    A, B = _proj2(h, wrt0, wct0, _pack8(b10))

    for l, (p, wr, wa, nxt) in enumerate([
        (g0, wr0, wa0, (wrt1, wct1, b11)),
        (g1, wr1, wa1, (wrtq, wctq, b1q)),
    ]):
        aux_e = _pack8(wr, wa, p["e_b2"], p["a_w"][0], p["a_b"])
        w2t = p["e_w2"].T
        Ps = []
        for sl in range(NSLICE):
            G = _gather_add(A, B, row, col, sl)
            ef = _gcl_edge(G, radial, ea, w2t, aux_e, sl)
            Ps.append(_scatter_add(ef, row, zeros128, sl))
        h, A, B = _node(
            h, Ps[0], Ps[1],
            p["n_w1"][:, :H].T, p["n_w1"][:, H:].T, p["n_w2"].T,
            nxt[0], nxt[1], _pack8(p["n_b1"], p["n_b2"], nxt[2]))

    aux_q = _pack8(wrq, waq, q["b2"], q["w3"][0])
    w2tq = q["w2"].T
    parts = []
    for sl in range(NSLICE):
        G = _gather_add(A, B, row, col, sl)
        s = _eq_edge(G, radial, ea, w2tq, aux_q, sl)
        parts.append(
            _eq_scatter(s.reshape(ES), row, col, xs0, xs1, xs2, zeros_n, sl))
    comps = []
    for kk in range(3):
        tot = sum(p[kk][:N] + p[kk][N:] for p in parts)
        comps.append(tot)
    aggx = jnp.stack(comps, axis=1)
    x_out = x + aggx * (1.0 / NORM_FACTOR)
    return h, x_out
